# Initial kernel scaffold; baseline (speedup 1.0000x reference)
import jax
import jax.numpy as jnp
from jax.experimental import pallas as pl


def kernel(atom_fea, edge_fea, r_ij, dist, edge_index, triplet_idx, W_fc, b_fc, bn1_w, bn1_b, bn2_w, bn2_b):
    raise NotImplementedError("write your pallas kernel here")


# speedup vs baseline: 16.4549x; 16.4549x over previous
import functools

import jax
import jax.numpy as jnp
import numpy as np
from jax import lax
from jax.experimental import pallas as pl
from jax.experimental.pallas import tpu as pltpu
from jax.experimental.pallas import tpu_sc as plsc

N = 10000
E = 320000
T = 640000
AF = 128
EPW = 20
OUT1 = 256
IN_DIM = 176

NC = 2
NS = 16
NW = NC * NS
BW = T // NW
CH = 80
ITERS = BW // CH

BT = 1024
GT = T // BT
BE = 3200
ROWS_PER_TILE = N // NS


def _ang_xr(e1, e2):
    ef1 = e1[:, :16]
    ef2 = e2[:, :16]
    p = e1[:, 16:20] * e2[:, 16:20]
    cos = (p[:, 0:1] + p[:, 1:2] + p[:, 2:3]) / p[:, 3:4]
    cos = jnp.clip(cos, -1.0, 1.0)
    centers = (lax.broadcasted_iota(jnp.int32, (1, 16), 1).astype(jnp.float32)
               * (2.0 / 15.0) - 1.0)
    ang = jnp.exp(-((cos - centers) ** 2) / (0.15 ** 2))
    return jnp.concatenate([ef1, ef2, ang], axis=1)


def _prep_body(ef_r, r_r, d_r, out_r):
    out_r[...] = jnp.concatenate(
        [ef_r[...], r_r[...], jnp.maximum(d_r[...], 1e-8)], axis=1)


def _prep(edge_fea, r_ij, dist2d):
    return pl.pallas_call(
        _prep_body,
        grid=(E // BE,),
        in_specs=[
            pl.BlockSpec((BE, 16), lambda i: (i, 0)),
            pl.BlockSpec((BE, 3), lambda i: (i, 0)),
            pl.BlockSpec((BE, 1), lambda i: (i, 0)),
        ],
        out_specs=pl.BlockSpec((BE, EPW), lambda i: (i, 0)),
        out_shape=jax.ShapeDtypeStruct((E, EPW), jnp.float32),
    )(edge_fea, r_ij, dist2d)


def _sc_gather(t1, t2, dst, epack, atom_fea):
    mesh = plsc.VectorSubcoreMesh(core_axis_name="c", subcore_axis_name="s")

    @functools.partial(
        pl.kernel,
        out_type=(
            jax.ShapeDtypeStruct((T, AF), jnp.float32),
            jax.ShapeDtypeStruct((T, EPW), jnp.float32),
            jax.ShapeDtypeStruct((T, EPW), jnp.float32),
            jax.ShapeDtypeStruct((T,), jnp.int32),
        ),
        mesh=mesh,
        scratch_types=[
            pltpu.VMEM((CH,), jnp.int32),
            pltpu.VMEM((CH,), jnp.int32),
            pltpu.VMEM((CH,), jnp.int32),
            pltpu.VMEM((CH, EPW), jnp.float32),
            pltpu.VMEM((CH, EPW), jnp.float32),
            pltpu.VMEM((CH, AF), jnp.float32),
            pltpu.SemaphoreType.DMA,
            pltpu.SemaphoreType.DMA,
            pltpu.SemaphoreType.DMA,
        ],
        compiler_params=pltpu.CompilerParams(use_tc_tiling_on_sc=False),
    )
    def g(t1_h, t2_h, dst_h, ep_h, af_h, xa_o, ep1_o, ep2_o, c_o,
          i1_v, i2_v, c_v, e1_v, e2_v, xa_v, s_a, s_b, s_c):
        wid = lax.axis_index("s") * NC + lax.axis_index("c")
        base0 = wid * BW

        def step(j, carry):
            base = base0 + j * CH
            pltpu.sync_copy(t1_h.at[pl.ds(base, CH)], i1_v)
            pltpu.sync_copy(t2_h.at[pl.ds(base, CH)], i2_v)
            pltpu.async_copy(dst_h.at[i1_v], c_v, s_a).wait()
            cpa = pltpu.async_copy(af_h.at[c_v], xa_v, s_a)
            cp1 = pltpu.async_copy(ep_h.at[i1_v], e1_v, s_b)
            cp2 = pltpu.async_copy(ep_h.at[i2_v], e2_v, s_c)
            cp1.wait()
            cp2.wait()
            cpa.wait()
            pltpu.sync_copy(c_v, c_o.at[pl.ds(base, CH)])
            pltpu.sync_copy(xa_v, xa_o.at[pl.ds(base, CH)])
            pltpu.sync_copy(e1_v, ep1_o.at[pl.ds(base, CH)])
            pltpu.sync_copy(e2_v, ep2_o.at[pl.ds(base, CH)])
            return carry

        lax.fori_loop(0, ITERS, step, 0)

    return g(t1, t2, dst, epack, atom_fea)


def _stats_body(xa_r, ep1_r, ep2_r, wt_r, b_r, g1_r, be1_r, wst_o, bs_o,
                maa, mar, mrr, sa, sr):
    i = pl.program_id(0)

    @pl.when(i == 0)
    def _():
        maa[...] = jnp.zeros_like(maa)
        mar[...] = jnp.zeros_like(mar)
        mrr[...] = jnp.zeros_like(mrr)
        sa[...] = jnp.zeros_like(sa)
        sr[...] = jnp.zeros_like(sr)

    xa_b = xa_r[...]
    xr_b = _ang_xr(ep1_r[...], ep2_r[...])
    dn = (((0,), (0,)), ((), ()))
    maa[...] += lax.dot_general(xa_b, xa_b, dn, preferred_element_type=jnp.float32)
    mar[...] += lax.dot_general(xa_b, xr_b, dn, preferred_element_type=jnp.float32)
    mrr[...] += lax.dot_general(xr_b, xr_b, dn, preferred_element_type=jnp.float32)
    sa[...] += jnp.sum(xa_b, axis=0, keepdims=True)
    sr[...] += jnp.sum(xr_b, axis=0, keepdims=True)

    @pl.when(i == GT - 1)
    def _():
        wt = wt_r[...]
        wa = wt[:AF, :]
        wr = wt[AF:, :]
        dn0 = (((0,), (0,)), ((), ()))
        z_top = (jnp.dot(maa[...], wa, preferred_element_type=jnp.float32)
                 + jnp.dot(mar[...], wr, preferred_element_type=jnp.float32))
        z_bot = (lax.dot_general(mar[...], wa, dn0, preferred_element_type=jnp.float32)
                 + jnp.dot(mrr[...], wr, preferred_element_type=jnp.float32))
        sw = (jnp.dot(sa[...], wa, preferred_element_type=jnp.float32)
              + jnp.dot(sr[...], wr, preferred_element_type=jnp.float32))
        bvec = b_r[...]
        tf = jnp.float32(T)
        e2 = (jnp.sum(wa * z_top, axis=0, keepdims=True)
              + jnp.sum(wr * z_bot, axis=0, keepdims=True)
              + 2.0 * bvec * sw + tf * bvec * bvec)
        mean = sw / tf + bvec
        var = e2 / tf - mean * mean
        s1 = g1_r[...] / jnp.sqrt(var + 1e-5)
        t1 = be1_r[...] - mean * s1
        wst_o[...] = wt * s1
        bs_o[...] = bvec * s1 + t1


def _stats(xa, ep1, ep2, wt, b2d, g12d, be12d):
    return pl.pallas_call(
        _stats_body,
        grid=(GT,),
        in_specs=[
            pl.BlockSpec((BT, AF), lambda i: (i, 0)),
            pl.BlockSpec((BT, EPW), lambda i: (i, 0)),
            pl.BlockSpec((BT, EPW), lambda i: (i, 0)),
            pl.BlockSpec((IN_DIM, OUT1), lambda i: (0, 0)),
            pl.BlockSpec((1, OUT1), lambda i: (0, 0)),
            pl.BlockSpec((1, OUT1), lambda i: (0, 0)),
            pl.BlockSpec((1, OUT1), lambda i: (0, 0)),
        ],
        out_specs=[
            pl.BlockSpec((IN_DIM, OUT1), lambda i: (0, 0)),
            pl.BlockSpec((1, OUT1), lambda i: (0, 0)),
        ],
        out_shape=[
            jax.ShapeDtypeStruct((IN_DIM, OUT1), jnp.float32),
            jax.ShapeDtypeStruct((1, OUT1), jnp.float32),
        ],
        scratch_shapes=[
            pltpu.VMEM((AF, AF), jnp.float32),
            pltpu.VMEM((AF, 48), jnp.float32),
            pltpu.VMEM((48, 48), jnp.float32),
            pltpu.VMEM((1, AF), jnp.float32),
            pltpu.VMEM((1, 48), jnp.float32),
        ],
    )(xa, ep1, ep2, wt, b2d, g12d, be12d)


def _fc_body(xa_r, ep1_r, ep2_r, wst_r, bs_r, msg_o):
    xa_b = xa_r[...]
    xr_b = _ang_xr(ep1_r[...], ep2_r[...])
    wt = wst_r[...]
    y = (jnp.dot(xa_b, wt[:AF, :], preferred_element_type=jnp.float32)
         + jnp.dot(xr_b, wt[AF:, :], preferred_element_type=jnp.float32)
         + bs_r[...])
    gate = y[:, :AF]
    core = y[:, AF:]
    sig = 1.0 / (1.0 + jnp.exp(-gate))
    sp = jnp.maximum(core, 0.0) + jnp.log1p(jnp.exp(-jnp.abs(core)))
    msg_o[...] = sig * sp


def _fc(xa, ep1, ep2, wst, bs):
    return pl.pallas_call(
        _fc_body,
        grid=(GT,),
        in_specs=[
            pl.BlockSpec((BT, AF), lambda i: (i, 0)),
            pl.BlockSpec((BT, EPW), lambda i: (i, 0)),
            pl.BlockSpec((BT, EPW), lambda i: (i, 0)),
            pl.BlockSpec((IN_DIM, OUT1), lambda i: (0, 0)),
            pl.BlockSpec((1, OUT1), lambda i: (0, 0)),
        ],
        out_specs=pl.BlockSpec((BT, AF), lambda i: (i, 0)),
        out_shape=jax.ShapeDtypeStruct((T, AF), jnp.float32),
    )(xa, ep1, ep2, wst, bs)


def _sc_scatter(msg, cidx, zeros):
    mesh = plsc.VectorSubcoreMesh(core_axis_name="c", subcore_axis_name="s")

    @functools.partial(
        pl.kernel,
        out_type=jax.ShapeDtypeStruct((2 * N, AF), jnp.float32),
        mesh=mesh,
        scratch_types=[
            pltpu.VMEM((CH,), jnp.int32),
            pltpu.VMEM((CH, AF), jnp.float32),
            pltpu.VMEM_SHARED((N, AF), jnp.float32),
        ],
        compiler_params=pltpu.CompilerParams(use_tc_tiling_on_sc=False),
    )
    def r(msg_h, c_h, z_h, out_h, idx_v, msg_v, acc_sh):
        cid = lax.axis_index("c")
        sid = lax.axis_index("s")
        wid = cid * NS + sid
        pltpu.sync_copy(z_h.at[pl.ds(sid * ROWS_PER_TILE, ROWS_PER_TILE)],
                        acc_sh.at[pl.ds(sid * ROWS_PER_TILE, ROWS_PER_TILE)])
        plsc.subcore_barrier()
        base0 = wid * BW

        def step(j, carry):
            base = base0 + j * CH
            pltpu.sync_copy(c_h.at[pl.ds(base, CH)], idx_v)
            pltpu.sync_copy(msg_h.at[pl.ds(base, CH)], msg_v)
            pltpu.sync_copy(msg_v, acc_sh.at[idx_v], add=True)
            return carry

        lax.fori_loop(0, ITERS, step, 0)
        plsc.subcore_barrier()
        pltpu.sync_copy(
            acc_sh.at[pl.ds(sid * ROWS_PER_TILE, ROWS_PER_TILE)],
            out_h.at[pl.ds(cid * N + sid * ROWS_PER_TILE, ROWS_PER_TILE)])

    return r(msg, cidx, zeros)


def _final_body(a0_r, a1_r, af_r, w2_r, b2_r, out_r):
    agg = a0_r[...] + a1_r[...]
    m = jnp.mean(agg, axis=0, keepdims=True)
    d = agg - m
    v = jnp.mean(d * d, axis=0, keepdims=True)
    nrm = d / jnp.sqrt(v + 1e-5) * w2_r[...] + b2_r[...]
    x = af_r[...] + nrm
    out_r[...] = jnp.maximum(x, 0.0) + jnp.log1p(jnp.exp(-jnp.abs(x)))


def _final(a0, a1, atom_fea, w22d, b22d):
    return pl.pallas_call(
        _final_body,
        grid=(1,),
        in_specs=[
            pl.BlockSpec((N, AF), lambda i: (0, 0)),
            pl.BlockSpec((N, AF), lambda i: (0, 0)),
            pl.BlockSpec((N, AF), lambda i: (0, 0)),
            pl.BlockSpec((1, AF), lambda i: (0, 0)),
            pl.BlockSpec((1, AF), lambda i: (0, 0)),
        ],
        out_specs=pl.BlockSpec((N, AF), lambda i: (0, 0)),
        out_shape=jax.ShapeDtypeStruct((N, AF), jnp.float32),
    )(a0, a1, atom_fea, w22d, b22d)


def kernel(atom_fea, edge_fea, r_ij, dist, edge_index, triplet_idx,
           W_fc, b_fc, bn1_w, bn1_b, bn2_w, bn2_b):
    t1 = triplet_idx[0].astype(jnp.int32)
    t2 = triplet_idx[1].astype(jnp.int32)
    dst = edge_index[1].astype(jnp.int32)
    wt = jnp.transpose(W_fc)
    b2d = b_fc.reshape(1, OUT1)
    g12d = bn1_w.reshape(1, OUT1)
    be12d = bn1_b.reshape(1, OUT1)
    w22d = bn2_w.reshape(1, AF)
    b22d = bn2_b.reshape(1, AF)
    zeros = jnp.zeros((N, AF), jnp.float32)

    epack = _prep(edge_fea, r_ij, dist.reshape(E, 1))
    xa, ep1, ep2, cidx = _sc_gather(t1, t2, dst, epack, atom_fea)
    wst, bs = _stats(xa, ep1, ep2, wt, b2d, g12d, be12d)
    msg = _fc(xa, ep1, ep2, wst, bs)
    agg = _sc_scatter(msg, cidx, zeros)
    return _final(agg[:N], agg[N:], atom_fea, w22d, b22d)


# speedup vs baseline: 22.5170x; 1.3684x over previous
import functools

import jax
import jax.numpy as jnp
import numpy as np
from jax import lax
from jax.experimental import pallas as pl
from jax.experimental.pallas import tpu as pltpu
from jax.experimental.pallas import tpu_sc as plsc

N = 10000
E = 320000
T = 640000
AF = 128
EPW = 20
OUT1 = 256
IN_DIM = 176

NC = 2
NS = 16
NW = NC * NS
BW = T // NW
CH = 80
ITERS = BW // CH
UNR = 5

BT = 3200
GT = T // BT
BE = 3200
ROWS_PER_TILE = N // NS


def _ang_xr(e1, e2):
    ef1 = e1[:, :16]
    ef2 = e2[:, :16]
    p = e1[:, 16:20] * e2[:, 16:20]
    cos = (p[:, 0:1] + p[:, 1:2] + p[:, 2:3]) / p[:, 3:4]
    cos = jnp.clip(cos, -1.0, 1.0)
    centers = (lax.broadcasted_iota(jnp.int32, (1, 16), 1).astype(jnp.float32)
               * (2.0 / 15.0) - 1.0)
    ang = jnp.exp(-((cos - centers) ** 2) / (0.15 ** 2))
    return jnp.concatenate([ef1, ef2, ang], axis=1)


def _prep_body(ef_r, r_r, d_r, out_r):
    out_r[...] = jnp.concatenate(
        [ef_r[...], r_r[...], jnp.maximum(d_r[...], 1e-8)], axis=1)


def _prep(edge_fea, r_ij, dist2d):
    return pl.pallas_call(
        _prep_body,
        grid=(E // BE,),
        in_specs=[
            pl.BlockSpec((BE, 16), lambda i: (i, 0)),
            pl.BlockSpec((BE, 3), lambda i: (i, 0)),
            pl.BlockSpec((BE, 1), lambda i: (i, 0)),
        ],
        out_specs=pl.BlockSpec((BE, EPW), lambda i: (i, 0)),
        out_shape=jax.ShapeDtypeStruct((E, EPW), jnp.float32),
    )(edge_fea, r_ij, dist2d)


def _sc_gather(t1, t2, dst, epack, atom_fea):
    mesh = plsc.VectorSubcoreMesh(core_axis_name="c", subcore_axis_name="s")

    @functools.partial(
        pl.kernel,
        out_type=(
            jax.ShapeDtypeStruct((T, AF), jnp.float32),
            jax.ShapeDtypeStruct((T, EPW), jnp.float32),
            jax.ShapeDtypeStruct((T, EPW), jnp.float32),
            jax.ShapeDtypeStruct((T,), jnp.int32),
        ),
        mesh=mesh,
        scratch_types=[
            pltpu.VMEM((UNR, CH), jnp.int32),
            pltpu.VMEM((UNR, CH), jnp.int32),
            pltpu.VMEM((UNR, CH), jnp.int32),
            pltpu.VMEM((UNR, CH, EPW), jnp.float32),
            pltpu.VMEM((UNR, CH, EPW), jnp.float32),
            pltpu.VMEM((UNR, CH, AF), jnp.float32),
            [pltpu.SemaphoreType.DMA] * UNR,
            [pltpu.SemaphoreType.DMA] * UNR,
        ],
        compiler_params=pltpu.CompilerParams(use_tc_tiling_on_sc=False),
    )
    def g(t1_h, t2_h, dst_h, ep_h, af_h, xa_o, ep1_o, ep2_o, c_o,
          i1_v, i2_v, c_v, e1_v, e2_v, xa_v, sems_a, sems_b):
        wid = lax.axis_index("s") * NC + lax.axis_index("c")
        base0 = wid * BW

        def step(jo, carry):
            jbase = base0 + jo * (UNR * CH)
            h_idx = []
            for k in range(UNR):
                b = jbase + k * CH
                h1 = pltpu.async_copy(t1_h.at[pl.ds(b, CH)], i1_v.at[k], sems_a[k])
                h2 = pltpu.async_copy(t2_h.at[pl.ds(b, CH)], i2_v.at[k], sems_a[k])
                h_idx.append((h1, h2))
            h_c = []
            for k in range(UNR):
                h_idx[k][0].wait()
                h_idx[k][1].wait()
                h_c.append(pltpu.async_copy(dst_h.at[i1_v.at[k]], c_v.at[k], sems_a[k]))
            h_g = []
            for k in range(UNR):
                h_c[k].wait()
                ga = pltpu.async_copy(af_h.at[c_v.at[k]], xa_v.at[k], sems_a[k])
                g1 = pltpu.async_copy(ep_h.at[i1_v.at[k]], e1_v.at[k], sems_b[k])
                g2 = pltpu.async_copy(ep_h.at[i2_v.at[k]], e2_v.at[k], sems_b[k])
                h_g.append((ga, g1, g2))
            h_w = []
            for k in range(UNR):
                b = jbase + k * CH
                for h in h_g[k]:
                    h.wait()
                w0 = pltpu.async_copy(c_v.at[k], c_o.at[pl.ds(b, CH)], sems_a[k])
                w1 = pltpu.async_copy(xa_v.at[k], xa_o.at[pl.ds(b, CH)], sems_a[k])
                w2 = pltpu.async_copy(e1_v.at[k], ep1_o.at[pl.ds(b, CH)], sems_b[k])
                w3 = pltpu.async_copy(e2_v.at[k], ep2_o.at[pl.ds(b, CH)], sems_b[k])
                h_w.append((w0, w1, w2, w3))
            for k in range(UNR):
                for h in h_w[k]:
                    h.wait()
            return carry

        lax.fori_loop(0, ITERS // UNR, step, 0)

    return g(t1, t2, dst, epack, atom_fea)


def _stats_body(xa_r, ep1_r, ep2_r, wt_r, b_r, g1_r, be1_r, wst_o, bs_o,
                maa, mar, mrr, sa, sr):
    i = pl.program_id(0)

    @pl.when(i == 0)
    def _():
        maa[...] = jnp.zeros_like(maa)
        mar[...] = jnp.zeros_like(mar)
        mrr[...] = jnp.zeros_like(mrr)
        sa[...] = jnp.zeros_like(sa)
        sr[...] = jnp.zeros_like(sr)

    xa_b = xa_r[...]
    xr_b = _ang_xr(ep1_r[...], ep2_r[...])
    dn = (((0,), (0,)), ((), ()))
    maa[...] += lax.dot_general(xa_b, xa_b, dn, preferred_element_type=jnp.float32)
    mar[...] += lax.dot_general(xa_b, xr_b, dn, preferred_element_type=jnp.float32)
    mrr[...] += lax.dot_general(xr_b, xr_b, dn, preferred_element_type=jnp.float32)
    sa[...] += jnp.sum(xa_b, axis=0, keepdims=True)
    sr[...] += jnp.sum(xr_b, axis=0, keepdims=True)

    @pl.when(i == GT - 1)
    def _():
        wt = wt_r[...]
        wa = wt[:AF, :]
        wr = wt[AF:, :]
        dn0 = (((0,), (0,)), ((), ()))
        z_top = (jnp.dot(maa[...], wa, preferred_element_type=jnp.float32)
                 + jnp.dot(mar[...], wr, preferred_element_type=jnp.float32))
        z_bot = (lax.dot_general(mar[...], wa, dn0, preferred_element_type=jnp.float32)
                 + jnp.dot(mrr[...], wr, preferred_element_type=jnp.float32))
        sw = (jnp.dot(sa[...], wa, preferred_element_type=jnp.float32)
              + jnp.dot(sr[...], wr, preferred_element_type=jnp.float32))
        bvec = b_r[...]
        tf = jnp.float32(T)
        e2 = (jnp.sum(wa * z_top, axis=0, keepdims=True)
              + jnp.sum(wr * z_bot, axis=0, keepdims=True)
              + 2.0 * bvec * sw + tf * bvec * bvec)
        mean = sw / tf + bvec
        var = e2 / tf - mean * mean
        s1 = g1_r[...] / jnp.sqrt(var + 1e-5)
        t1 = be1_r[...] - mean * s1
        wst_o[...] = wt * s1
        bs_o[...] = bvec * s1 + t1


def _stats(xa, ep1, ep2, wt, b2d, g12d, be12d):
    return pl.pallas_call(
        _stats_body,
        grid=(GT,),
        in_specs=[
            pl.BlockSpec((BT, AF), lambda i: (i, 0)),
            pl.BlockSpec((BT, EPW), lambda i: (i, 0)),
            pl.BlockSpec((BT, EPW), lambda i: (i, 0)),
            pl.BlockSpec((IN_DIM, OUT1), lambda i: (0, 0)),
            pl.BlockSpec((1, OUT1), lambda i: (0, 0)),
            pl.BlockSpec((1, OUT1), lambda i: (0, 0)),
            pl.BlockSpec((1, OUT1), lambda i: (0, 0)),
        ],
        out_specs=[
            pl.BlockSpec((IN_DIM, OUT1), lambda i: (0, 0)),
            pl.BlockSpec((1, OUT1), lambda i: (0, 0)),
        ],
        out_shape=[
            jax.ShapeDtypeStruct((IN_DIM, OUT1), jnp.float32),
            jax.ShapeDtypeStruct((1, OUT1), jnp.float32),
        ],
        scratch_shapes=[
            pltpu.VMEM((AF, AF), jnp.float32),
            pltpu.VMEM((AF, 48), jnp.float32),
            pltpu.VMEM((48, 48), jnp.float32),
            pltpu.VMEM((1, AF), jnp.float32),
            pltpu.VMEM((1, 48), jnp.float32),
        ],
    )(xa, ep1, ep2, wt, b2d, g12d, be12d)


def _fc_body(xa_r, ep1_r, ep2_r, wst_r, bs_r, msg_o):
    xa_b = xa_r[...]
    xr_b = _ang_xr(ep1_r[...], ep2_r[...])
    wt = wst_r[...]
    y = (jnp.dot(xa_b, wt[:AF, :], preferred_element_type=jnp.float32)
         + jnp.dot(xr_b, wt[AF:, :], preferred_element_type=jnp.float32)
         + bs_r[...])
    gate = y[:, :AF]
    core = y[:, AF:]
    sig = 1.0 / (1.0 + jnp.exp(-gate))
    sp = jnp.maximum(core, 0.0) + jnp.log1p(jnp.exp(-jnp.abs(core)))
    msg_o[...] = sig * sp


def _fc(xa, ep1, ep2, wst, bs):
    return pl.pallas_call(
        _fc_body,
        grid=(GT,),
        in_specs=[
            pl.BlockSpec((BT, AF), lambda i: (i, 0)),
            pl.BlockSpec((BT, EPW), lambda i: (i, 0)),
            pl.BlockSpec((BT, EPW), lambda i: (i, 0)),
            pl.BlockSpec((IN_DIM, OUT1), lambda i: (0, 0)),
            pl.BlockSpec((1, OUT1), lambda i: (0, 0)),
        ],
        out_specs=pl.BlockSpec((BT, AF), lambda i: (i, 0)),
        out_shape=jax.ShapeDtypeStruct((T, AF), jnp.float32),
    )(xa, ep1, ep2, wst, bs)


def _sc_scatter(msg, cidx, zeros):
    mesh = plsc.VectorSubcoreMesh(core_axis_name="c", subcore_axis_name="s")

    HAF = AF // 2
    BWS = T // NS
    ITERS_S = BWS // CH

    @functools.partial(
        pl.kernel,
        out_type=jax.ShapeDtypeStruct((N, AF), jnp.float32),
        mesh=mesh,
        scratch_types=[
            pltpu.VMEM((UNR, CH), jnp.int32),
            pltpu.VMEM((UNR, CH, HAF), jnp.float32),
            pltpu.VMEM_SHARED((N, HAF), jnp.float32),
            [pltpu.SemaphoreType.DMA] * UNR,
            [pltpu.SemaphoreType.DMA] * UNR,
        ],
        compiler_params=pltpu.CompilerParams(use_tc_tiling_on_sc=False),
    )
    def r(msg_h, c_h, z_h, out_h, idx_v, msg_v, acc_sh, sems_l, sems_s):
        cid = lax.axis_index("c")
        sid = lax.axis_index("s")
        col0 = cid * HAF
        pltpu.sync_copy(z_h.at[pl.ds(sid * ROWS_PER_TILE, ROWS_PER_TILE)],
                        acc_sh.at[pl.ds(sid * ROWS_PER_TILE, ROWS_PER_TILE)])
        plsc.subcore_barrier()
        base0 = sid * BWS

        def step(jo, carry):
            jbase = base0 + jo * (UNR * CH)
            h_l = []
            for k in range(UNR):
                b = jbase + k * CH
                l0 = pltpu.async_copy(c_h.at[pl.ds(b, CH)], idx_v.at[k], sems_l[k])
                l1 = pltpu.async_copy(msg_h.at[pl.ds(b, CH), pl.ds(col0, HAF)],
                                      msg_v.at[k], sems_l[k])
                h_l.append((l0, l1))
            h_s = []
            for k in range(UNR):
                h_l[k][0].wait()
                h_l[k][1].wait()
                h_s.append(pltpu.async_copy(
                    msg_v.at[k], acc_sh.at[idx_v.at[k]], sems_s[k], add=True))
            for k in range(UNR):
                h_s[k].wait()
            return carry

        lax.fori_loop(0, ITERS_S // UNR, step, 0)
        plsc.subcore_barrier()
        pltpu.sync_copy(
            acc_sh.at[pl.ds(sid * ROWS_PER_TILE, ROWS_PER_TILE)],
            out_h.at[pl.ds(sid * ROWS_PER_TILE, ROWS_PER_TILE), pl.ds(col0, HAF)])

    return r(msg, cidx, zeros)


def _final_body(a0_r, af_r, w2_r, b2_r, out_r):
    agg = a0_r[...]
    m = jnp.mean(agg, axis=0, keepdims=True)
    d = agg - m
    v = jnp.mean(d * d, axis=0, keepdims=True)
    nrm = d / jnp.sqrt(v + 1e-5) * w2_r[...] + b2_r[...]
    x = af_r[...] + nrm
    out_r[...] = jnp.maximum(x, 0.0) + jnp.log1p(jnp.exp(-jnp.abs(x)))


def _final(a0, atom_fea, w22d, b22d):
    return pl.pallas_call(
        _final_body,
        grid=(1,),
        in_specs=[
            pl.BlockSpec((N, AF), lambda i: (0, 0)),
            pl.BlockSpec((N, AF), lambda i: (0, 0)),
            pl.BlockSpec((1, AF), lambda i: (0, 0)),
            pl.BlockSpec((1, AF), lambda i: (0, 0)),
        ],
        out_specs=pl.BlockSpec((N, AF), lambda i: (0, 0)),
        out_shape=jax.ShapeDtypeStruct((N, AF), jnp.float32),
    )(a0, atom_fea, w22d, b22d)


def kernel(atom_fea, edge_fea, r_ij, dist, edge_index, triplet_idx,
           W_fc, b_fc, bn1_w, bn1_b, bn2_w, bn2_b):
    t1 = triplet_idx[0].astype(jnp.int32)
    t2 = triplet_idx[1].astype(jnp.int32)
    dst = edge_index[1].astype(jnp.int32)
    wt = jnp.transpose(W_fc)
    b2d = b_fc.reshape(1, OUT1)
    g12d = bn1_w.reshape(1, OUT1)
    be12d = bn1_b.reshape(1, OUT1)
    w22d = bn2_w.reshape(1, AF)
    b22d = bn2_b.reshape(1, AF)
    zeros = jnp.zeros((N, AF // 2), jnp.float32)

    epack = _prep(edge_fea, r_ij, dist.reshape(E, 1))
    xa, ep1, ep2, cidx = _sc_gather(t1, t2, dst, epack, atom_fea)
    wst, bs = _stats(xa, ep1, ep2, wt, b2d, g12d, be12d)
    msg = _fc(xa, ep1, ep2, wst, bs)
    agg = _sc_scatter(msg, cidx, zeros)
    return _final(agg, atom_fea, w22d, b22d)


# speedup vs baseline: 26.1330x; 1.1606x over previous
import functools

import jax
import jax.numpy as jnp
import numpy as np
from jax import lax
from jax.experimental import pallas as pl
from jax.experimental.pallas import tpu as pltpu
from jax.experimental.pallas import tpu_sc as plsc

N = 10000
E = 320000
T = 640000
AF = 128
EPW = 20
EPS = 24
OUT1 = 256
IN_DIM = 176

NC = 2
NS = 16
NW = NC * NS
BW = T // NW
CH = 80
ITERS = BW // CH
UNR = 5

BT = 3200
GT = T // BT
BE = 3200
ROWS_PER_TILE = N // NS


def _ang_xr(b):
    p = b[:, 16:19] * b[:, 40:43]
    num = p[:, 0:1] + p[:, 1:2] + p[:, 2:3]
    den = jnp.maximum(b[:, 19:20], 1e-8) * jnp.maximum(b[:, 43:44], 1e-8)
    cos = jnp.clip(num / den, -1.0, 1.0)
    centers = (lax.broadcasted_iota(jnp.int32, (1, 16), 1).astype(jnp.float32)
               * (2.0 / 15.0) - 1.0)
    ang = jnp.exp(-((cos - centers) ** 2) / (0.15 ** 2))
    return jnp.concatenate([b[:, 0:16], b[:, 24:40], ang], axis=1)


def _sc_repack(epack128):
    mesh = plsc.VectorSubcoreMesh(core_axis_name="c", subcore_axis_name="s")
    EW = E // NW
    RCH = 1000

    @functools.partial(
        pl.kernel,
        out_type=jax.ShapeDtypeStruct((E, EPS), jnp.float32),
        mesh=mesh,
        scratch_types=[
            pltpu.VMEM((2, RCH, EPS), jnp.float32),
            [pltpu.SemaphoreType.DMA] * 2,
        ],
        compiler_params=pltpu.CompilerParams(use_tc_tiling_on_sc=False),
    )
    def rp(ep_h, out_h, buf_v, sems):
        wid = lax.axis_index("s") * NC + lax.axis_index("c")
        base0 = wid * EW

        def step(jo, carry):
            hs = []
            for k in range(2):
                b = base0 + (2 * jo + k) * RCH
                hs.append(pltpu.async_copy(
                    ep_h.at[pl.ds(b, RCH), pl.ds(0, EPS)], buf_v.at[k], sems[k]))
            ws = []
            for k in range(2):
                b = base0 + (2 * jo + k) * RCH
                hs[k].wait()
                ws.append(pltpu.async_copy(
                    buf_v.at[k], out_h.at[pl.ds(b, RCH)], sems[k]))
            for k in range(2):
                ws[k].wait()
            return carry

        lax.fori_loop(0, EW // (2 * RCH), step, 0)

    return rp(epack128)


def _sc_gather(t1, t2, dst, epack, atom_fea):
    mesh = plsc.VectorSubcoreMesh(core_axis_name="c", subcore_axis_name="s")

    @functools.partial(
        pl.kernel,
        out_type=(
            jax.ShapeDtypeStruct((T, AF), jnp.float32),
            jax.ShapeDtypeStruct((T, AF), jnp.float32),
            jax.ShapeDtypeStruct((T,), jnp.int32),
        ),
        mesh=mesh,
        scratch_types=[
            pltpu.VMEM((UNR, CH), jnp.int32),
            pltpu.VMEM((UNR, CH), jnp.int32),
            pltpu.VMEM((UNR, CH), jnp.int32),
            pltpu.VMEM((UNR, CH, EPS), jnp.float32),
            pltpu.VMEM((UNR, CH, EPS), jnp.float32),
            pltpu.VMEM((UNR, CH, AF), jnp.float32),
            [pltpu.SemaphoreType.DMA] * UNR,
            [pltpu.SemaphoreType.DMA] * UNR,
        ],
        compiler_params=pltpu.CompilerParams(use_tc_tiling_on_sc=False),
    )
    def g(t1_h, t2_h, dst_h, ep_h, af_h, xa_o, epc_o, c_o,
          i1_v, i2_v, c_v, e1_v, e2_v, xa_v, sems_a, sems_b):
        wid = lax.axis_index("s") * NC + lax.axis_index("c")
        base0 = wid * BW

        def step(jo, carry):
            jbase = base0 + jo * (UNR * CH)
            h_idx = []
            for k in range(UNR):
                b = jbase + k * CH
                h1 = pltpu.async_copy(t1_h.at[pl.ds(b, CH)], i1_v.at[k], sems_a[k])
                h2 = pltpu.async_copy(t2_h.at[pl.ds(b, CH)], i2_v.at[k], sems_a[k])
                h_idx.append((h1, h2))
            h_c = []
            for k in range(UNR):
                h_idx[k][0].wait()
                h_idx[k][1].wait()
                h_c.append(pltpu.async_copy(dst_h.at[i1_v.at[k]], c_v.at[k], sems_a[k]))
            h_g = []
            for k in range(UNR):
                h_c[k].wait()
                ga = pltpu.async_copy(af_h.at[c_v.at[k]], xa_v.at[k], sems_a[k])
                g1 = pltpu.async_copy(ep_h.at[i1_v.at[k]], e1_v.at[k], sems_b[k])
                g2 = pltpu.async_copy(ep_h.at[i2_v.at[k]], e2_v.at[k], sems_b[k])
                h_g.append((ga, g1, g2))
            h_w = []
            for k in range(UNR):
                b = jbase + k * CH
                for h in h_g[k]:
                    h.wait()
                w0 = pltpu.async_copy(c_v.at[k], c_o.at[pl.ds(b, CH)], sems_a[k])
                w1 = pltpu.async_copy(xa_v.at[k], xa_o.at[pl.ds(b, CH)], sems_a[k])
                w2 = pltpu.async_copy(e1_v.at[k],
                                      epc_o.at[pl.ds(b, CH), pl.ds(0, EPS)],
                                      sems_b[k])
                w3 = pltpu.async_copy(e2_v.at[k],
                                      epc_o.at[pl.ds(b, CH), pl.ds(EPS, EPS)],
                                      sems_b[k])
                h_w.append((w0, w1, w2, w3))
            for k in range(UNR):
                for h in h_w[k]:
                    h.wait()
            return carry

        lax.fori_loop(0, ITERS // UNR, step, 0)

    return g(t1, t2, dst, epack, atom_fea)


_X_SPECS = [
    pl.BlockSpec((BT, AF), lambda i: (i, 0)),
    pl.BlockSpec((BT, AF), lambda i: (i, 0)),
]


def _stats_body(xa_r, epc_r, wt_r, b_r, g1_r, be1_r, wst_o, bs_o,
                maa, mar, mrr, sa, sr):
    i = pl.program_id(0)

    @pl.when(i == 0)
    def _():
        maa[...] = jnp.zeros_like(maa)
        mar[...] = jnp.zeros_like(mar)
        mrr[...] = jnp.zeros_like(mrr)
        sa[...] = jnp.zeros_like(sa)
        sr[...] = jnp.zeros_like(sr)

    xa_b = xa_r[...]
    xr_b = _ang_xr(epc_r[...])
    dn = (((0,), (0,)), ((), ()))
    maa[...] += lax.dot_general(xa_b, xa_b, dn, preferred_element_type=jnp.float32)
    mar[...] += lax.dot_general(xa_b, xr_b, dn, preferred_element_type=jnp.float32)
    mrr[...] += lax.dot_general(xr_b, xr_b, dn, preferred_element_type=jnp.float32)
    sa[...] += jnp.sum(xa_b, axis=0, keepdims=True)
    sr[...] += jnp.sum(xr_b, axis=0, keepdims=True)

    @pl.when(i == GT - 1)
    def _():
        wt = wt_r[...]
        wa = wt[:AF, :]
        wr = wt[AF:, :]
        dn0 = (((0,), (0,)), ((), ()))
        z_top = (jnp.dot(maa[...], wa, preferred_element_type=jnp.float32)
                 + jnp.dot(mar[...], wr, preferred_element_type=jnp.float32))
        z_bot = (lax.dot_general(mar[...], wa, dn0, preferred_element_type=jnp.float32)
                 + jnp.dot(mrr[...], wr, preferred_element_type=jnp.float32))
        sw = (jnp.dot(sa[...], wa, preferred_element_type=jnp.float32)
              + jnp.dot(sr[...], wr, preferred_element_type=jnp.float32))
        bvec = b_r[...]
        tf = jnp.float32(T)
        e2 = (jnp.sum(wa * z_top, axis=0, keepdims=True)
              + jnp.sum(wr * z_bot, axis=0, keepdims=True)
              + 2.0 * bvec * sw + tf * bvec * bvec)
        mean = sw / tf + bvec
        var = e2 / tf - mean * mean
        s1 = g1_r[...] / jnp.sqrt(var + 1e-5)
        t1 = be1_r[...] - mean * s1
        wst_o[...] = wt * s1
        bs_o[...] = bvec * s1 + t1


def _stats(xa, epc, wt, b2d, g12d, be12d):
    return pl.pallas_call(
        _stats_body,
        grid=(GT,),
        in_specs=[
            pl.BlockSpec((BT, AF), lambda i: (i, 0)),
            pl.BlockSpec((BT, AF), lambda i: (i, 0)),
            pl.BlockSpec((IN_DIM, OUT1), lambda i: (0, 0)),
            pl.BlockSpec((1, OUT1), lambda i: (0, 0)),
            pl.BlockSpec((1, OUT1), lambda i: (0, 0)),
            pl.BlockSpec((1, OUT1), lambda i: (0, 0)),
        ],
        out_specs=[
            pl.BlockSpec((IN_DIM, OUT1), lambda i: (0, 0)),
            pl.BlockSpec((1, OUT1), lambda i: (0, 0)),
        ],
        out_shape=[
            jax.ShapeDtypeStruct((IN_DIM, OUT1), jnp.float32),
            jax.ShapeDtypeStruct((1, OUT1), jnp.float32),
        ],
        scratch_shapes=[
            pltpu.VMEM((AF, AF), jnp.float32),
            pltpu.VMEM((AF, 48), jnp.float32),
            pltpu.VMEM((48, 48), jnp.float32),
            pltpu.VMEM((1, AF), jnp.float32),
            pltpu.VMEM((1, 48), jnp.float32),
        ],
    )(xa, epc, wt, b2d, g12d, be12d)


def _fc_body(xa_r, epc_r, wst_r, bs_r, msg_o):
    xa_b = xa_r[...]
    xr_b = _ang_xr(epc_r[...])
    wt = wst_r[...]
    y = (jnp.dot(xa_b, wt[:AF, :], preferred_element_type=jnp.float32)
         + jnp.dot(xr_b, wt[AF:, :], preferred_element_type=jnp.float32)
         + bs_r[...])
    gate = y[:, :AF]
    core = y[:, AF:]
    sig = 1.0 / (1.0 + jnp.exp(-gate))
    sp = jnp.maximum(core, 0.0) + jnp.log1p(jnp.exp(-jnp.abs(core)))
    msg_o[...] = sig * sp


def _fc(xa, epc, wst, bs):
    return pl.pallas_call(
        _fc_body,
        grid=(GT,),
        in_specs=[
            pl.BlockSpec((BT, AF), lambda i: (i, 0)),
            pl.BlockSpec((BT, AF), lambda i: (i, 0)),
            pl.BlockSpec((IN_DIM, OUT1), lambda i: (0, 0)),
            pl.BlockSpec((1, OUT1), lambda i: (0, 0)),
        ],
        out_specs=pl.BlockSpec((BT, AF), lambda i: (i, 0)),
        out_shape=jax.ShapeDtypeStruct((T, AF), jnp.float32),
    )(xa, epc, wst, bs)


def _sc_scatter(msg, cidx, zeros):
    mesh = plsc.VectorSubcoreMesh(core_axis_name="c", subcore_axis_name="s")

    HAF = AF // 2
    BWS = T // NS
    ITERS_S = BWS // CH

    @functools.partial(
        pl.kernel,
        out_type=jax.ShapeDtypeStruct((N, AF), jnp.float32),
        mesh=mesh,
        scratch_types=[
            pltpu.VMEM((UNR, CH), jnp.int32),
            pltpu.VMEM((UNR, CH, HAF), jnp.float32),
            pltpu.VMEM_SHARED((N, HAF), jnp.float32),
            [pltpu.SemaphoreType.DMA] * UNR,
            [pltpu.SemaphoreType.DMA] * UNR,
        ],
        compiler_params=pltpu.CompilerParams(use_tc_tiling_on_sc=False),
    )
    def r(msg_h, c_h, z_h, out_h, idx_v, msg_v, acc_sh, sems_l, sems_s):
        cid = lax.axis_index("c")
        sid = lax.axis_index("s")
        col0 = cid * HAF
        pltpu.sync_copy(z_h.at[pl.ds(sid * ROWS_PER_TILE, ROWS_PER_TILE)],
                        acc_sh.at[pl.ds(sid * ROWS_PER_TILE, ROWS_PER_TILE)])
        plsc.subcore_barrier()
        base0 = sid * BWS

        def step(jo, carry):
            jbase = base0 + jo * (UNR * CH)
            h_l = []
            for k in range(UNR):
                b = jbase + k * CH
                l0 = pltpu.async_copy(c_h.at[pl.ds(b, CH)], idx_v.at[k], sems_l[k])
                l1 = pltpu.async_copy(msg_h.at[pl.ds(b, CH), pl.ds(col0, HAF)],
                                      msg_v.at[k], sems_l[k])
                h_l.append((l0, l1))
            h_s = []
            for k in range(UNR):
                h_l[k][0].wait()
                h_l[k][1].wait()
                h_s.append(pltpu.async_copy(
                    msg_v.at[k], acc_sh.at[idx_v.at[k]], sems_s[k], add=True))
            for k in range(UNR):
                h_s[k].wait()
            return carry

        lax.fori_loop(0, ITERS_S // UNR, step, 0)
        plsc.subcore_barrier()
        pltpu.sync_copy(
            acc_sh.at[pl.ds(sid * ROWS_PER_TILE, ROWS_PER_TILE)],
            out_h.at[pl.ds(sid * ROWS_PER_TILE, ROWS_PER_TILE), pl.ds(col0, HAF)])

    return r(msg, cidx, zeros)


def _final_body(a0_r, af_r, w2_r, b2_r, out_r):
    agg = a0_r[...]
    m = jnp.mean(agg, axis=0, keepdims=True)
    d = agg - m
    v = jnp.mean(d * d, axis=0, keepdims=True)
    nrm = d / jnp.sqrt(v + 1e-5) * w2_r[...] + b2_r[...]
    x = af_r[...] + nrm
    out_r[...] = jnp.maximum(x, 0.0) + jnp.log1p(jnp.exp(-jnp.abs(x)))


def _final(a0, atom_fea, w22d, b22d):
    return pl.pallas_call(
        _final_body,
        grid=(1,),
        in_specs=[
            pl.BlockSpec((N, AF), lambda i: (0, 0)),
            pl.BlockSpec((N, AF), lambda i: (0, 0)),
            pl.BlockSpec((1, AF), lambda i: (0, 0)),
            pl.BlockSpec((1, AF), lambda i: (0, 0)),
        ],
        out_specs=pl.BlockSpec((N, AF), lambda i: (0, 0)),
        out_shape=jax.ShapeDtypeStruct((N, AF), jnp.float32),
    )(a0, atom_fea, w22d, b22d)


def kernel(atom_fea, edge_fea, r_ij, dist, edge_index, triplet_idx,
           W_fc, b_fc, bn1_w, bn1_b, bn2_w, bn2_b):
    t1 = triplet_idx[0].astype(jnp.int32)
    t2 = triplet_idx[1].astype(jnp.int32)
    dst = edge_index[1].astype(jnp.int32)
    wt = jnp.transpose(W_fc)
    b2d = b_fc.reshape(1, OUT1)
    g12d = bn1_w.reshape(1, OUT1)
    be12d = bn1_b.reshape(1, OUT1)
    w22d = bn2_w.reshape(1, AF)
    b22d = bn2_b.reshape(1, AF)
    zeros = jnp.zeros((N, AF // 2), jnp.float32)

    epack128 = jnp.concatenate(
        [edge_fea, r_ij, dist.reshape(E, 1),
         jnp.zeros((E, AF - EPW), jnp.float32)], axis=1)
    ep24 = _sc_repack(epack128)
    xa, epc, cidx = _sc_gather(t1, t2, dst, ep24, atom_fea)
    wst, bs = _stats(xa, epc, wt, b2d, g12d, be12d)
    msg = _fc(xa, epc, wst, bs)
    agg = _sc_scatter(msg, cidx, zeros)
    return _final(agg, atom_fea, w22d, b22d)


# speedup vs baseline: 34.3066x; 1.3128x over previous
import functools

import jax
import jax.numpy as jnp
import numpy as np
from jax import lax
from jax.experimental import pallas as pl
from jax.experimental.pallas import tpu as pltpu
from jax.experimental.pallas import tpu_sc as plsc

N = 10000
E = 320000
T = 640000
AF = 128
EPW = 20
EPS = 24
OUT1 = 256
IN_DIM = 176

NC = 2
NS = 16
NW = NC * NS
BW = T // NW
CH = 80
ITERS = BW // CH
UNR = 5

BT = 3200
GT = T // BT
BE = 3200
ROWS_PER_TILE = N // NS


def _bz_ang(b, sperm, m1, m2):
    lanes = lax.broadcasted_iota(jnp.int32, (1, 128), 1)
    bz = jnp.where(lanes < 48, b, 0.0)
    bshift = jnp.dot(bz, sperm, preferred_element_type=jnp.float32)
    p = bz * bshift
    num = jnp.dot(p, m1, preferred_element_type=jnp.float32)
    den = jnp.dot(p, m2, preferred_element_type=jnp.float32)
    cos = jnp.clip(num / jnp.maximum(den, 1e-16), -1.0, 1.0)
    centers = (lax.broadcasted_iota(jnp.int32, (1, 16), 1).astype(jnp.float32)
               * (2.0 / 15.0) - 1.0)
    ang = jnp.exp(-((cos - centers) ** 2) / (0.15 ** 2))
    return bz, ang


def _sc_repack(epack128):
    mesh = plsc.VectorSubcoreMesh(core_axis_name="c", subcore_axis_name="s")
    EW = E // NW
    RCH = 1000

    @functools.partial(
        pl.kernel,
        out_type=jax.ShapeDtypeStruct((E, EPS), jnp.float32),
        mesh=mesh,
        scratch_types=[
            pltpu.VMEM((2, RCH, EPS), jnp.float32),
            [pltpu.SemaphoreType.DMA] * 2,
        ],
        compiler_params=pltpu.CompilerParams(use_tc_tiling_on_sc=False),
    )
    def rp(ep_h, out_h, buf_v, sems):
        wid = lax.axis_index("s") * NC + lax.axis_index("c")
        base0 = wid * EW

        def step(jo, carry):
            hs = []
            for k in range(2):
                b = base0 + (2 * jo + k) * RCH
                hs.append(pltpu.async_copy(
                    ep_h.at[pl.ds(b, RCH), pl.ds(0, EPS)], buf_v.at[k], sems[k]))
            ws = []
            for k in range(2):
                b = base0 + (2 * jo + k) * RCH
                hs[k].wait()
                ws.append(pltpu.async_copy(
                    buf_v.at[k], out_h.at[pl.ds(b, RCH)], sems[k]))
            for k in range(2):
                ws[k].wait()
            return carry

        lax.fori_loop(0, EW // (2 * RCH), step, 0)

    return rp(epack128)


def _sc_gather(t1, t2, dst, epack, atom_fea):
    mesh = plsc.VectorSubcoreMesh(core_axis_name="c", subcore_axis_name="s")

    @functools.partial(
        pl.kernel,
        out_type=(
            jax.ShapeDtypeStruct((T, AF), jnp.float32),
            jax.ShapeDtypeStruct((T, AF), jnp.float32),
            jax.ShapeDtypeStruct((T,), jnp.int32),
        ),
        mesh=mesh,
        scratch_types=[
            pltpu.VMEM((UNR, CH), jnp.int32),
            pltpu.VMEM((UNR, CH), jnp.int32),
            pltpu.VMEM((UNR, CH), jnp.int32),
            pltpu.VMEM((UNR, CH, EPS), jnp.float32),
            pltpu.VMEM((UNR, CH, EPS), jnp.float32),
            pltpu.VMEM((UNR, CH, AF), jnp.float32),
            [pltpu.SemaphoreType.DMA] * UNR,
            [pltpu.SemaphoreType.DMA] * UNR,
        ],
        compiler_params=pltpu.CompilerParams(use_tc_tiling_on_sc=False),
    )
    def g(t1_h, t2_h, dst_h, ep_h, af_h, xa_o, epc_o, c_o,
          i1_v, i2_v, c_v, e1_v, e2_v, xa_v, sems_a, sems_b):
        wid = lax.axis_index("s") * NC + lax.axis_index("c")
        base0 = wid * BW

        def step(jo, carry):
            jbase = base0 + jo * (UNR * CH)
            h_idx = []
            for k in range(UNR):
                b = jbase + k * CH
                h1 = pltpu.async_copy(t1_h.at[pl.ds(b, CH)], i1_v.at[k], sems_a[k])
                h2 = pltpu.async_copy(t2_h.at[pl.ds(b, CH)], i2_v.at[k], sems_a[k])
                h_idx.append((h1, h2))
            h_c = []
            for k in range(UNR):
                h_idx[k][0].wait()
                h_idx[k][1].wait()
                h_c.append(pltpu.async_copy(dst_h.at[i1_v.at[k]], c_v.at[k], sems_a[k]))
            h_g = []
            for k in range(UNR):
                h_c[k].wait()
                ga = pltpu.async_copy(af_h.at[c_v.at[k]], xa_v.at[k], sems_a[k])
                g1 = pltpu.async_copy(ep_h.at[i1_v.at[k]], e1_v.at[k], sems_b[k])
                g2 = pltpu.async_copy(ep_h.at[i2_v.at[k]], e2_v.at[k], sems_b[k])
                h_g.append((ga, g1, g2))
            h_w = []
            for k in range(UNR):
                b = jbase + k * CH
                for h in h_g[k]:
                    h.wait()
                w0 = pltpu.async_copy(c_v.at[k], c_o.at[pl.ds(b, CH)], sems_a[k])
                w1 = pltpu.async_copy(xa_v.at[k], xa_o.at[pl.ds(b, CH)], sems_a[k])
                w2 = pltpu.async_copy(e1_v.at[k],
                                      epc_o.at[pl.ds(b, CH), pl.ds(0, EPS)],
                                      sems_b[k])
                w3 = pltpu.async_copy(e2_v.at[k],
                                      epc_o.at[pl.ds(b, CH), pl.ds(EPS, EPS)],
                                      sems_b[k])
                h_w.append((w0, w1, w2, w3))
            for k in range(UNR):
                for h in h_w[k]:
                    h.wait()
            return carry

        lax.fori_loop(0, ITERS // UNR, step, 0)

    return g(t1, t2, dst, epack, atom_fea)


_X_SPECS = [
    pl.BlockSpec((BT, AF), lambda i: (i, 0)),
    pl.BlockSpec((BT, AF), lambda i: (i, 0)),
]


def _stats_body(xa_r, epc_r, sperm_r, m1_r, m2_r, wt_r, wa_r, we_r, wg_r,
                b_r, g1_r, be1_r, wsa_o, wse_o, wsg_o, bs_o,
                maa, mar, mrr, sa, sr):
    i = pl.program_id(0)

    @pl.when(i == 0)
    def _():
        maa[...] = jnp.zeros_like(maa)
        mar[...] = jnp.zeros_like(mar)
        mrr[...] = jnp.zeros_like(mrr)
        sa[...] = jnp.zeros_like(sa)
        sr[...] = jnp.zeros_like(sr)

    xa_b = xa_r[...]
    bz, ang = _bz_ang(epc_r[...], sperm_r[...], m1_r[...], m2_r[...])
    xr_b = jnp.concatenate([bz[:, 0:16], bz[:, 24:40], ang], axis=1)
    dn = (((0,), (0,)), ((), ()))
    maa[...] += lax.dot_general(xa_b, xa_b, dn, preferred_element_type=jnp.float32)
    mar[...] += lax.dot_general(xa_b, xr_b, dn, preferred_element_type=jnp.float32)
    mrr[...] += lax.dot_general(xr_b, xr_b, dn, preferred_element_type=jnp.float32)
    sa[...] += jnp.sum(xa_b, axis=0, keepdims=True)
    sr[...] += jnp.sum(xr_b, axis=0, keepdims=True)

    @pl.when(i == GT - 1)
    def _():
        wt = wt_r[...]
        wa = wt[:AF, :]
        wr = wt[AF:, :]
        dn0 = (((0,), (0,)), ((), ()))
        z_top = (jnp.dot(maa[...], wa, preferred_element_type=jnp.float32)
                 + jnp.dot(mar[...], wr, preferred_element_type=jnp.float32))
        z_bot = (lax.dot_general(mar[...], wa, dn0, preferred_element_type=jnp.float32)
                 + jnp.dot(mrr[...], wr, preferred_element_type=jnp.float32))
        sw = (jnp.dot(sa[...], wa, preferred_element_type=jnp.float32)
              + jnp.dot(sr[...], wr, preferred_element_type=jnp.float32))
        bvec = b_r[...]
        tf = jnp.float32(T)
        e2 = (jnp.sum(wa * z_top, axis=0, keepdims=True)
              + jnp.sum(wr * z_bot, axis=0, keepdims=True)
              + 2.0 * bvec * sw + tf * bvec * bvec)
        mean = sw / tf + bvec
        var = e2 / tf - mean * mean
        s1 = g1_r[...] / jnp.sqrt(var + 1e-5)
        t1 = be1_r[...] - mean * s1
        wsa_o[...] = wa_r[...] * s1
        wse_o[...] = we_r[...] * s1
        wsg_o[...] = wg_r[...] * s1
        bs_o[...] = bvec * s1 + t1


def _stats(xa, epc, sperm, m1, m2, wt, wa, we, wg, b2d, g12d, be12d):
    return pl.pallas_call(
        _stats_body,
        grid=(GT,),
        in_specs=[
            pl.BlockSpec((BT, AF), lambda i: (i, 0)),
            pl.BlockSpec((BT, AF), lambda i: (i, 0)),
            pl.BlockSpec((AF, AF), lambda i: (0, 0)),
            pl.BlockSpec((AF, 16), lambda i: (0, 0)),
            pl.BlockSpec((AF, 16), lambda i: (0, 0)),
            pl.BlockSpec((IN_DIM, OUT1), lambda i: (0, 0)),
            pl.BlockSpec((AF, OUT1), lambda i: (0, 0)),
            pl.BlockSpec((AF, OUT1), lambda i: (0, 0)),
            pl.BlockSpec((16, OUT1), lambda i: (0, 0)),
            pl.BlockSpec((1, OUT1), lambda i: (0, 0)),
            pl.BlockSpec((1, OUT1), lambda i: (0, 0)),
            pl.BlockSpec((1, OUT1), lambda i: (0, 0)),
        ],
        out_specs=[
            pl.BlockSpec((AF, OUT1), lambda i: (0, 0)),
            pl.BlockSpec((AF, OUT1), lambda i: (0, 0)),
            pl.BlockSpec((16, OUT1), lambda i: (0, 0)),
            pl.BlockSpec((1, OUT1), lambda i: (0, 0)),
        ],
        out_shape=[
            jax.ShapeDtypeStruct((AF, OUT1), jnp.float32),
            jax.ShapeDtypeStruct((AF, OUT1), jnp.float32),
            jax.ShapeDtypeStruct((16, OUT1), jnp.float32),
            jax.ShapeDtypeStruct((1, OUT1), jnp.float32),
        ],
        scratch_shapes=[
            pltpu.VMEM((AF, AF), jnp.float32),
            pltpu.VMEM((AF, 48), jnp.float32),
            pltpu.VMEM((48, 48), jnp.float32),
            pltpu.VMEM((1, AF), jnp.float32),
            pltpu.VMEM((1, 48), jnp.float32),
        ],
    )(xa, epc, sperm, m1, m2, wt, wa, we, wg, b2d, g12d, be12d)


def _fc_body(xa_r, epc_r, sperm_r, m1_r, m2_r, wsa_r, wse_r, wsg_r, bs_r, msg_o):
    xa_b = xa_r[...]
    bz, ang = _bz_ang(epc_r[...], sperm_r[...], m1_r[...], m2_r[...])
    y = (jnp.dot(xa_b, wsa_r[...], preferred_element_type=jnp.float32)
         + jnp.dot(bz, wse_r[...], preferred_element_type=jnp.float32)
         + jnp.dot(ang, wsg_r[...], preferred_element_type=jnp.float32)
         + bs_r[...])
    gate = y[:, :AF]
    core = y[:, AF:]
    sig = 1.0 / (1.0 + jnp.exp(-gate))
    sp = jnp.maximum(core, 0.0) + jnp.log1p(jnp.exp(-jnp.abs(core)))
    msg_o[...] = sig * sp


def _fc(xa, epc, sperm, m1, m2, wsa, wse, wsg, bs):
    return pl.pallas_call(
        _fc_body,
        grid=(GT,),
        in_specs=[
            pl.BlockSpec((BT, AF), lambda i: (i, 0)),
            pl.BlockSpec((BT, AF), lambda i: (i, 0)),
            pl.BlockSpec((AF, AF), lambda i: (0, 0)),
            pl.BlockSpec((AF, 16), lambda i: (0, 0)),
            pl.BlockSpec((AF, 16), lambda i: (0, 0)),
            pl.BlockSpec((AF, OUT1), lambda i: (0, 0)),
            pl.BlockSpec((AF, OUT1), lambda i: (0, 0)),
            pl.BlockSpec((16, OUT1), lambda i: (0, 0)),
            pl.BlockSpec((1, OUT1), lambda i: (0, 0)),
        ],
        out_specs=pl.BlockSpec((BT, AF), lambda i: (i, 0)),
        out_shape=jax.ShapeDtypeStruct((T, AF), jnp.float32),
    )(xa, epc, sperm, m1, m2, wsa, wse, wsg, bs)


def _sc_scatter(msg, cidx, zeros):
    mesh = plsc.VectorSubcoreMesh(core_axis_name="c", subcore_axis_name="s")

    HAF = AF // 2
    BWS = T // NS
    ITERS_S = BWS // CH

    @functools.partial(
        pl.kernel,
        out_type=jax.ShapeDtypeStruct((N, AF), jnp.float32),
        mesh=mesh,
        scratch_types=[
            pltpu.VMEM((UNR, CH), jnp.int32),
            pltpu.VMEM((UNR, CH, HAF), jnp.float32),
            pltpu.VMEM_SHARED((N, HAF), jnp.float32),
            [pltpu.SemaphoreType.DMA] * UNR,
            [pltpu.SemaphoreType.DMA] * UNR,
        ],
        compiler_params=pltpu.CompilerParams(use_tc_tiling_on_sc=False),
    )
    def r(msg_h, c_h, z_h, out_h, idx_v, msg_v, acc_sh, sems_l, sems_s):
        cid = lax.axis_index("c")
        sid = lax.axis_index("s")
        col0 = cid * HAF
        pltpu.sync_copy(z_h.at[pl.ds(sid * ROWS_PER_TILE, ROWS_PER_TILE)],
                        acc_sh.at[pl.ds(sid * ROWS_PER_TILE, ROWS_PER_TILE)])
        plsc.subcore_barrier()
        base0 = sid * BWS

        def step(jo, carry):
            jbase = base0 + jo * (UNR * CH)
            h_l = []
            for k in range(UNR):
                b = jbase + k * CH
                l0 = pltpu.async_copy(c_h.at[pl.ds(b, CH)], idx_v.at[k], sems_l[k])
                l1 = pltpu.async_copy(msg_h.at[pl.ds(b, CH), pl.ds(col0, HAF)],
                                      msg_v.at[k], sems_l[k])
                h_l.append((l0, l1))
            h_s = []
            for k in range(UNR):
                h_l[k][0].wait()
                h_l[k][1].wait()
                h_s.append(pltpu.async_copy(
                    msg_v.at[k], acc_sh.at[idx_v.at[k]], sems_s[k], add=True))
            for k in range(UNR):
                h_s[k].wait()
            return carry

        lax.fori_loop(0, ITERS_S // UNR, step, 0)
        plsc.subcore_barrier()
        pltpu.sync_copy(
            acc_sh.at[pl.ds(sid * ROWS_PER_TILE, ROWS_PER_TILE)],
            out_h.at[pl.ds(sid * ROWS_PER_TILE, ROWS_PER_TILE), pl.ds(col0, HAF)])

    return r(msg, cidx, zeros)


def _final_body(a0_r, af_r, w2_r, b2_r, out_r):
    agg = a0_r[...]
    m = jnp.mean(agg, axis=0, keepdims=True)
    d = agg - m
    v = jnp.mean(d * d, axis=0, keepdims=True)
    nrm = d / jnp.sqrt(v + 1e-5) * w2_r[...] + b2_r[...]
    x = af_r[...] + nrm
    out_r[...] = jnp.maximum(x, 0.0) + jnp.log1p(jnp.exp(-jnp.abs(x)))


def _final(a0, atom_fea, w22d, b22d):
    return pl.pallas_call(
        _final_body,
        grid=(1,),
        in_specs=[
            pl.BlockSpec((N, AF), lambda i: (0, 0)),
            pl.BlockSpec((N, AF), lambda i: (0, 0)),
            pl.BlockSpec((1, AF), lambda i: (0, 0)),
            pl.BlockSpec((1, AF), lambda i: (0, 0)),
        ],
        out_specs=pl.BlockSpec((N, AF), lambda i: (0, 0)),
        out_shape=jax.ShapeDtypeStruct((N, AF), jnp.float32),
    )(a0, atom_fea, w22d, b22d)


def kernel(atom_fea, edge_fea, r_ij, dist, edge_index, triplet_idx,
           W_fc, b_fc, bn1_w, bn1_b, bn2_w, bn2_b):
    t1 = triplet_idx[0].astype(jnp.int32)
    t2 = triplet_idx[1].astype(jnp.int32)
    dst = edge_index[1].astype(jnp.int32)
    wt = jnp.transpose(W_fc)
    b2d = b_fc.reshape(1, OUT1)
    g12d = bn1_w.reshape(1, OUT1)
    be12d = bn1_b.reshape(1, OUT1)
    w22d = bn2_w.reshape(1, AF)
    b22d = bn2_b.reshape(1, AF)
    zeros = jnp.zeros((N, AF // 2), jnp.float32)

    epack128 = jnp.concatenate(
        [edge_fea, r_ij, dist.reshape(E, 1),
         jnp.zeros((E, AF - EPW), jnp.float32)], axis=1)
    sperm_np = np.zeros((AF, AF), np.float32)
    for i in range(4):
        sperm_np[40 + i, 16 + i] = 1.0
    m1_np = np.zeros((AF, 16), np.float32)
    m1_np[16:19, :] = 1.0
    m2_np = np.zeros((AF, 16), np.float32)
    m2_np[19, :] = 1.0
    sperm = jnp.asarray(sperm_np)
    m1 = jnp.asarray(m1_np)
    m2 = jnp.asarray(m2_np)
    wa = wt[:AF, :]
    we = (jnp.zeros((AF, OUT1), jnp.float32)
          .at[0:16].set(wt[AF:AF + 16, :])
          .at[24:40].set(wt[AF + 16:AF + 32, :]))
    wg = wt[AF + 32:, :]

    ep24 = _sc_repack(epack128)
    xa, epc, cidx = _sc_gather(t1, t2, dst, ep24, atom_fea)
    wsa, wse, wsg, bs = _stats(xa, epc, sperm, m1, m2, wt, wa, we, wg,
                               b2d, g12d, be12d)
    msg = _fc(xa, epc, sperm, m1, m2, wsa, wse, wsg, bs)
    agg = _sc_scatter(msg, cidx, zeros)
    return _final(agg, atom_fea, w22d, b22d)


# speedup vs baseline: 39.6632x; 1.1561x over previous
import functools

import jax
import jax.numpy as jnp
import numpy as np
from jax import lax
from jax.experimental import pallas as pl
from jax.experimental.pallas import tpu as pltpu
from jax.experimental.pallas import tpu_sc as plsc

N = 10000
E = 320000
T = 640000
AF = 128
EPW = 20
EPS = 24
OUT1 = 256
IN_DIM = 176

NC = 2
NS = 16
NW = NC * NS
BW = T // NW
CH = 80
ITERS = BW // CH
UNR = 5

BT = 3200
GT = T // BT
BE = 3200
ROWS_PER_TILE = N // NS


def _bz_ang(b, sperm, m1, m2):
    lanes = lax.broadcasted_iota(jnp.int32, (1, 128), 1)
    bz = jnp.where(lanes < 48, b, 0.0)
    bshift = jnp.dot(bz, sperm, preferred_element_type=jnp.float32)
    p = bz * bshift
    num = jnp.dot(p, m1, preferred_element_type=jnp.float32)
    den = jnp.dot(p, m2, preferred_element_type=jnp.float32)
    cos = jnp.clip(num / jnp.maximum(den, 1e-16), -1.0, 1.0)
    centers = (lax.broadcasted_iota(jnp.int32, (1, 16), 1).astype(jnp.float32)
               * (2.0 / 15.0) - 1.0)
    ang = jnp.exp(-((cos - centers) ** 2) / (0.15 ** 2))
    return bz, ang


def _prep_body(eft_r, rt_r, dt_r, out_r):
    stack = jnp.concatenate(
        [eft_r[...], rt_r[...], dt_r[...], jnp.zeros((4, BE), jnp.float32)],
        axis=0)
    t = jnp.transpose(stack)
    out_r[...] = jnp.concatenate([t, jnp.zeros((BE, AF - EPS), jnp.float32)],
                                 axis=1)


def _prep(eft, rt, dt):
    return pl.pallas_call(
        _prep_body,
        grid=(E // BE,),
        in_specs=[
            pl.BlockSpec((16, BE), lambda i: (0, i)),
            pl.BlockSpec((3, BE), lambda i: (0, i)),
            pl.BlockSpec((1, BE), lambda i: (0, i)),
        ],
        out_specs=pl.BlockSpec((BE, AF), lambda i: (i, 0)),
        out_shape=jax.ShapeDtypeStruct((E, AF), jnp.float32),
    )(eft, rt, dt)


def _sc_repack(epack128):
    mesh = plsc.VectorSubcoreMesh(core_axis_name="c", subcore_axis_name="s")
    EW = E // NW
    RCH = 1000

    @functools.partial(
        pl.kernel,
        out_type=jax.ShapeDtypeStruct((E, EPS), jnp.float32),
        mesh=mesh,
        scratch_types=[
            pltpu.VMEM((2, RCH, EPS), jnp.float32),
            [pltpu.SemaphoreType.DMA] * 2,
        ],
        compiler_params=pltpu.CompilerParams(use_tc_tiling_on_sc=False),
    )
    def rp(ep_h, out_h, buf_v, sems):
        wid = lax.axis_index("s") * NC + lax.axis_index("c")
        base0 = wid * EW

        def step(jo, carry):
            hs = []
            for k in range(2):
                b = base0 + (2 * jo + k) * RCH
                hs.append(pltpu.async_copy(
                    ep_h.at[pl.ds(b, RCH), pl.ds(0, EPS)], buf_v.at[k], sems[k]))
            ws = []
            for k in range(2):
                b = base0 + (2 * jo + k) * RCH
                hs[k].wait()
                ws.append(pltpu.async_copy(
                    buf_v.at[k], out_h.at[pl.ds(b, RCH)], sems[k]))
            for k in range(2):
                ws[k].wait()
            return carry

        lax.fori_loop(0, EW // (2 * RCH), step, 0)

    return rp(epack128)


def _sc_gather(t1, t2, dst, epack, atom_fea):
    mesh = plsc.VectorSubcoreMesh(core_axis_name="c", subcore_axis_name="s")

    @functools.partial(
        pl.kernel,
        out_type=(
            jax.ShapeDtypeStruct((T, AF), jnp.float32),
            jax.ShapeDtypeStruct((T, AF), jnp.float32),
            jax.ShapeDtypeStruct((T,), jnp.int32),
        ),
        mesh=mesh,
        scratch_types=[
            pltpu.VMEM((UNR, CH), jnp.int32),
            pltpu.VMEM((UNR, CH), jnp.int32),
            pltpu.VMEM((UNR, CH), jnp.int32),
            pltpu.VMEM((UNR, CH, EPS), jnp.float32),
            pltpu.VMEM((UNR, CH, EPS), jnp.float32),
            pltpu.VMEM((UNR, CH, AF), jnp.float32),
            [pltpu.SemaphoreType.DMA] * UNR,
            [pltpu.SemaphoreType.DMA] * UNR,
        ],
        compiler_params=pltpu.CompilerParams(use_tc_tiling_on_sc=False),
    )
    def g(t1_h, t2_h, dst_h, ep_h, af_h, xa_o, epc_o, c_o,
          i1_v, i2_v, c_v, e1_v, e2_v, xa_v, sems_a, sems_b):
        wid = lax.axis_index("s") * NC + lax.axis_index("c")
        base0 = wid * BW

        def step(jo, carry):
            jbase = base0 + jo * (UNR * CH)
            h_idx = []
            for k in range(UNR):
                b = jbase + k * CH
                h1 = pltpu.async_copy(t1_h.at[pl.ds(b, CH)], i1_v.at[k], sems_a[k])
                h2 = pltpu.async_copy(t2_h.at[pl.ds(b, CH)], i2_v.at[k], sems_a[k])
                h_idx.append((h1, h2))
            h_c = []
            for k in range(UNR):
                h_idx[k][0].wait()
                h_idx[k][1].wait()
                h_c.append(pltpu.async_copy(dst_h.at[i1_v.at[k]], c_v.at[k], sems_a[k]))
            h_g = []
            for k in range(UNR):
                h_c[k].wait()
                ga = pltpu.async_copy(af_h.at[c_v.at[k]], xa_v.at[k], sems_a[k])
                g1 = pltpu.async_copy(ep_h.at[i1_v.at[k]], e1_v.at[k], sems_b[k])
                g2 = pltpu.async_copy(ep_h.at[i2_v.at[k]], e2_v.at[k], sems_b[k])
                h_g.append((ga, g1, g2))
            h_w = []
            for k in range(UNR):
                b = jbase + k * CH
                for h in h_g[k]:
                    h.wait()
                w0 = pltpu.async_copy(c_v.at[k], c_o.at[pl.ds(b, CH)], sems_a[k])
                w1 = pltpu.async_copy(xa_v.at[k], xa_o.at[pl.ds(b, CH)], sems_a[k])
                w2 = pltpu.async_copy(e1_v.at[k],
                                      epc_o.at[pl.ds(b, CH), pl.ds(0, EPS)],
                                      sems_b[k])
                w3 = pltpu.async_copy(e2_v.at[k],
                                      epc_o.at[pl.ds(b, CH), pl.ds(EPS, EPS)],
                                      sems_b[k])
                h_w.append((w0, w1, w2, w3))
            for k in range(UNR):
                for h in h_w[k]:
                    h.wait()
            return carry

        lax.fori_loop(0, ITERS // UNR, step, 0)

    return g(t1, t2, dst, epack, atom_fea)


_X_SPECS = [
    pl.BlockSpec((BT, AF), lambda i: (i, 0)),
    pl.BlockSpec((BT, AF), lambda i: (i, 0)),
]


def _stats_body(xa_r, epc_r, sperm_r, m1_r, m2_r, wt_r, wa_r, we_r, wg_r,
                b_r, g1_r, be1_r, wsa_o, wse_o, wsg_o, bs_o,
                maa, mar, mrr, sa, sr):
    i = pl.program_id(0)

    @pl.when(i == 0)
    def _():
        maa[...] = jnp.zeros_like(maa)
        mar[...] = jnp.zeros_like(mar)
        mrr[...] = jnp.zeros_like(mrr)
        sa[...] = jnp.zeros_like(sa)
        sr[...] = jnp.zeros_like(sr)

    xa_b = xa_r[...]
    bz, ang = _bz_ang(epc_r[...], sperm_r[...], m1_r[...], m2_r[...])
    xr_b = jnp.concatenate([bz[:, 0:16], bz[:, 24:40], ang], axis=1)
    dn = (((0,), (0,)), ((), ()))
    maa[...] += lax.dot_general(xa_b, xa_b, dn, preferred_element_type=jnp.float32)
    mar[...] += lax.dot_general(xa_b, xr_b, dn, preferred_element_type=jnp.float32)
    mrr[...] += lax.dot_general(xr_b, xr_b, dn, preferred_element_type=jnp.float32)
    sa[...] += jnp.sum(xa_b, axis=0, keepdims=True)
    sr[...] += jnp.sum(xr_b, axis=0, keepdims=True)

    @pl.when(i == GT - 1)
    def _():
        wt = wt_r[...]
        wa = wt[:AF, :]
        wr = wt[AF:, :]
        dn0 = (((0,), (0,)), ((), ()))
        z_top = (jnp.dot(maa[...], wa, preferred_element_type=jnp.float32)
                 + jnp.dot(mar[...], wr, preferred_element_type=jnp.float32))
        z_bot = (lax.dot_general(mar[...], wa, dn0, preferred_element_type=jnp.float32)
                 + jnp.dot(mrr[...], wr, preferred_element_type=jnp.float32))
        sw = (jnp.dot(sa[...], wa, preferred_element_type=jnp.float32)
              + jnp.dot(sr[...], wr, preferred_element_type=jnp.float32))
        bvec = b_r[...]
        tf = jnp.float32(T)
        e2 = (jnp.sum(wa * z_top, axis=0, keepdims=True)
              + jnp.sum(wr * z_bot, axis=0, keepdims=True)
              + 2.0 * bvec * sw + tf * bvec * bvec)
        mean = sw / tf + bvec
        var = e2 / tf - mean * mean
        s1 = g1_r[...] / jnp.sqrt(var + 1e-5)
        t1 = be1_r[...] - mean * s1
        wsa_o[...] = wa_r[...] * s1
        wse_o[...] = we_r[...] * s1
        wsg_o[...] = wg_r[...] * s1
        bs_o[...] = bvec * s1 + t1


def _stats(xa, epc, sperm, m1, m2, wt, wa, we, wg, b2d, g12d, be12d):
    return pl.pallas_call(
        _stats_body,
        grid=(GT,),
        in_specs=[
            pl.BlockSpec((BT, AF), lambda i: (i, 0)),
            pl.BlockSpec((BT, AF), lambda i: (i, 0)),
            pl.BlockSpec((AF, AF), lambda i: (0, 0)),
            pl.BlockSpec((AF, 16), lambda i: (0, 0)),
            pl.BlockSpec((AF, 16), lambda i: (0, 0)),
            pl.BlockSpec((IN_DIM, OUT1), lambda i: (0, 0)),
            pl.BlockSpec((AF, OUT1), lambda i: (0, 0)),
            pl.BlockSpec((AF, OUT1), lambda i: (0, 0)),
            pl.BlockSpec((16, OUT1), lambda i: (0, 0)),
            pl.BlockSpec((1, OUT1), lambda i: (0, 0)),
            pl.BlockSpec((1, OUT1), lambda i: (0, 0)),
            pl.BlockSpec((1, OUT1), lambda i: (0, 0)),
        ],
        out_specs=[
            pl.BlockSpec((AF, OUT1), lambda i: (0, 0)),
            pl.BlockSpec((AF, OUT1), lambda i: (0, 0)),
            pl.BlockSpec((16, OUT1), lambda i: (0, 0)),
            pl.BlockSpec((1, OUT1), lambda i: (0, 0)),
        ],
        out_shape=[
            jax.ShapeDtypeStruct((AF, OUT1), jnp.float32),
            jax.ShapeDtypeStruct((AF, OUT1), jnp.float32),
            jax.ShapeDtypeStruct((16, OUT1), jnp.float32),
            jax.ShapeDtypeStruct((1, OUT1), jnp.float32),
        ],
        scratch_shapes=[
            pltpu.VMEM((AF, AF), jnp.float32),
            pltpu.VMEM((AF, 48), jnp.float32),
            pltpu.VMEM((48, 48), jnp.float32),
            pltpu.VMEM((1, AF), jnp.float32),
            pltpu.VMEM((1, 48), jnp.float32),
        ],
    )(xa, epc, sperm, m1, m2, wt, wa, we, wg, b2d, g12d, be12d)


def _fc_body(xa_r, epc_r, sperm_r, m1_r, m2_r, wsa_r, wse_r, wsg_r, bs_r, msg_o):
    xa_b = xa_r[...]
    bz, ang = _bz_ang(epc_r[...], sperm_r[...], m1_r[...], m2_r[...])
    y = (jnp.dot(xa_b, wsa_r[...], preferred_element_type=jnp.float32)
         + jnp.dot(bz, wse_r[...], preferred_element_type=jnp.float32)
         + jnp.dot(ang, wsg_r[...], preferred_element_type=jnp.float32)
         + bs_r[...])
    gate = y[:, :AF]
    core = y[:, AF:]
    sig = 1.0 / (1.0 + jnp.exp(-gate))
    sp = jnp.maximum(core, 0.0) + jnp.log1p(jnp.exp(-jnp.abs(core)))
    msg_o[...] = sig * sp


def _fc(xa, epc, sperm, m1, m2, wsa, wse, wsg, bs):
    return pl.pallas_call(
        _fc_body,
        grid=(GT,),
        in_specs=[
            pl.BlockSpec((BT, AF), lambda i: (i, 0)),
            pl.BlockSpec((BT, AF), lambda i: (i, 0)),
            pl.BlockSpec((AF, AF), lambda i: (0, 0)),
            pl.BlockSpec((AF, 16), lambda i: (0, 0)),
            pl.BlockSpec((AF, 16), lambda i: (0, 0)),
            pl.BlockSpec((AF, OUT1), lambda i: (0, 0)),
            pl.BlockSpec((AF, OUT1), lambda i: (0, 0)),
            pl.BlockSpec((16, OUT1), lambda i: (0, 0)),
            pl.BlockSpec((1, OUT1), lambda i: (0, 0)),
        ],
        out_specs=pl.BlockSpec((BT, AF), lambda i: (i, 0)),
        out_shape=jax.ShapeDtypeStruct((T, AF), jnp.float32),
    )(xa, epc, sperm, m1, m2, wsa, wse, wsg, bs)


def _sc_scatter(msg, cidx, zeros):
    mesh = plsc.VectorSubcoreMesh(core_axis_name="c", subcore_axis_name="s")

    HAF = AF // 2
    BWS = T // NS
    ITERS_S = BWS // CH

    @functools.partial(
        pl.kernel,
        out_type=jax.ShapeDtypeStruct((N, AF), jnp.float32),
        mesh=mesh,
        scratch_types=[
            pltpu.VMEM((UNR, CH), jnp.int32),
            pltpu.VMEM((UNR, CH, HAF), jnp.float32),
            pltpu.VMEM_SHARED((N, HAF), jnp.float32),
            [pltpu.SemaphoreType.DMA] * UNR,
            [pltpu.SemaphoreType.DMA] * UNR,
        ],
        compiler_params=pltpu.CompilerParams(use_tc_tiling_on_sc=False),
    )
    def r(msg_h, c_h, z_h, out_h, idx_v, msg_v, acc_sh, sems_l, sems_s):
        cid = lax.axis_index("c")
        sid = lax.axis_index("s")
        col0 = cid * HAF
        pltpu.sync_copy(z_h.at[pl.ds(sid * ROWS_PER_TILE, ROWS_PER_TILE)],
                        acc_sh.at[pl.ds(sid * ROWS_PER_TILE, ROWS_PER_TILE)])
        plsc.subcore_barrier()
        base0 = sid * BWS

        def step(jo, carry):
            jbase = base0 + jo * (UNR * CH)
            h_l = []
            for k in range(UNR):
                b = jbase + k * CH
                l0 = pltpu.async_copy(c_h.at[pl.ds(b, CH)], idx_v.at[k], sems_l[k])
                l1 = pltpu.async_copy(msg_h.at[pl.ds(b, CH), pl.ds(col0, HAF)],
                                      msg_v.at[k], sems_l[k])
                h_l.append((l0, l1))
            h_s = []
            for k in range(UNR):
                h_l[k][0].wait()
                h_l[k][1].wait()
                h_s.append(pltpu.async_copy(
                    msg_v.at[k], acc_sh.at[idx_v.at[k]], sems_s[k], add=True))
            for k in range(UNR):
                h_s[k].wait()
            return carry

        lax.fori_loop(0, ITERS_S // UNR, step, 0)
        plsc.subcore_barrier()
        pltpu.sync_copy(
            acc_sh.at[pl.ds(sid * ROWS_PER_TILE, ROWS_PER_TILE)],
            out_h.at[pl.ds(sid * ROWS_PER_TILE, ROWS_PER_TILE), pl.ds(col0, HAF)])

    return r(msg, cidx, zeros)


def _final_body(a0_r, af_r, w2_r, b2_r, out_r):
    agg = a0_r[...]
    m = jnp.mean(agg, axis=0, keepdims=True)
    d = agg - m
    v = jnp.mean(d * d, axis=0, keepdims=True)
    nrm = d / jnp.sqrt(v + 1e-5) * w2_r[...] + b2_r[...]
    x = af_r[...] + nrm
    out_r[...] = jnp.maximum(x, 0.0) + jnp.log1p(jnp.exp(-jnp.abs(x)))


def _final(a0, atom_fea, w22d, b22d):
    return pl.pallas_call(
        _final_body,
        grid=(1,),
        in_specs=[
            pl.BlockSpec((N, AF), lambda i: (0, 0)),
            pl.BlockSpec((N, AF), lambda i: (0, 0)),
            pl.BlockSpec((1, AF), lambda i: (0, 0)),
            pl.BlockSpec((1, AF), lambda i: (0, 0)),
        ],
        out_specs=pl.BlockSpec((N, AF), lambda i: (0, 0)),
        out_shape=jax.ShapeDtypeStruct((N, AF), jnp.float32),
    )(a0, atom_fea, w22d, b22d)


def kernel(atom_fea, edge_fea, r_ij, dist, edge_index, triplet_idx,
           W_fc, b_fc, bn1_w, bn1_b, bn2_w, bn2_b):
    t1 = triplet_idx[0].astype(jnp.int32)
    t2 = triplet_idx[1].astype(jnp.int32)
    dst = edge_index[1].astype(jnp.int32)
    wt = jnp.transpose(W_fc)
    b2d = b_fc.reshape(1, OUT1)
    g12d = bn1_w.reshape(1, OUT1)
    be12d = bn1_b.reshape(1, OUT1)
    w22d = bn2_w.reshape(1, AF)
    b22d = bn2_b.reshape(1, AF)
    zeros = jnp.zeros((N, AF // 2), jnp.float32)

    epack128 = _prep(jnp.transpose(edge_fea), jnp.transpose(r_ij),
                     dist.reshape(1, E))
    sperm_np = np.zeros((AF, AF), np.float32)
    for i in range(4):
        sperm_np[40 + i, 16 + i] = 1.0
    m1_np = np.zeros((AF, 16), np.float32)
    m1_np[16:19, :] = 1.0
    m2_np = np.zeros((AF, 16), np.float32)
    m2_np[19, :] = 1.0
    sperm = jnp.asarray(sperm_np)
    m1 = jnp.asarray(m1_np)
    m2 = jnp.asarray(m2_np)
    wa = wt[:AF, :]
    we = (jnp.zeros((AF, OUT1), jnp.float32)
          .at[0:16].set(wt[AF:AF + 16, :])
          .at[24:40].set(wt[AF + 16:AF + 32, :]))
    wg = wt[AF + 32:, :]

    ep24 = _sc_repack(epack128)
    xa, epc, cidx = _sc_gather(t1, t2, dst, ep24, atom_fea)
    wsa, wse, wsg, bs = _stats(xa, epc, sperm, m1, m2, wt, wa, we, wg,
                               b2d, g12d, be12d)
    msg = _fc(xa, epc, sperm, m1, m2, wsa, wse, wsg, bs)
    agg = _sc_scatter(msg, cidx, zeros)
    return _final(agg, atom_fea, w22d, b22d)


# speedup vs baseline: 44.9195x; 1.1325x over previous
import functools

import jax
import jax.numpy as jnp
import numpy as np
from jax import lax
from jax.experimental import pallas as pl
from jax.experimental.pallas import tpu as pltpu
from jax.experimental.pallas import tpu_sc as plsc

N = 10000
E = 320000
T = 640000
AF = 128
EPW = 20
EPS = 24
OUT1 = 256
IN_DIM = 176

NC = 2
NS = 16
NW = NC * NS
SLABS = 2
TSL = T // SLABS
BW = TSL // NW
CH = 80
ITERS = BW // CH
UNR = 5

BT = 3200
GT = TSL // BT
BE = 3200
ROWS_PER_TILE = N // NS


def _bz_ang(b, sperm, m1, m2):
    lanes = lax.broadcasted_iota(jnp.int32, (1, 128), 1)
    bz = jnp.where(lanes < 48, b, 0.0)
    bshift = jnp.dot(bz, sperm, preferred_element_type=jnp.float32)
    p = bz * bshift
    num = jnp.dot(p, m1, preferred_element_type=jnp.float32)
    den = jnp.dot(p, m2, preferred_element_type=jnp.float32)
    cos = jnp.clip(num / jnp.maximum(den, 1e-16), -1.0, 1.0)
    centers = (lax.broadcasted_iota(jnp.int32, (1, 16), 1).astype(jnp.float32)
               * (2.0 / 15.0) - 1.0)
    ang = jnp.exp(-((cos - centers) ** 2) / (0.15 ** 2))
    return bz, ang


def _prep_body(eft_r, rt_r, dt_r, out_r):
    stack = jnp.concatenate(
        [eft_r[...], rt_r[...], dt_r[...], jnp.zeros((4, BE), jnp.float32)],
        axis=0)
    t = jnp.transpose(stack)
    out_r[...] = jnp.concatenate([t, jnp.zeros((BE, AF - EPS), jnp.float32)],
                                 axis=1)


def _prep(eft, rt, dt):
    return pl.pallas_call(
        _prep_body,
        grid=(E // BE,),
        in_specs=[
            pl.BlockSpec((16, BE), lambda i: (0, i)),
            pl.BlockSpec((3, BE), lambda i: (0, i)),
            pl.BlockSpec((1, BE), lambda i: (0, i)),
        ],
        out_specs=pl.BlockSpec((BE, AF), lambda i: (i, 0)),
        out_shape=jax.ShapeDtypeStruct((E, AF), jnp.float32),
    )(eft, rt, dt)


def _sc_repack(epack128):
    mesh = plsc.VectorSubcoreMesh(core_axis_name="c", subcore_axis_name="s")
    EW = E // NW
    RCH = 1000

    @functools.partial(
        pl.kernel,
        out_type=jax.ShapeDtypeStruct((E, EPS), jnp.float32),
        mesh=mesh,
        scratch_types=[
            pltpu.VMEM((2, RCH, EPS), jnp.float32),
            [pltpu.SemaphoreType.DMA] * 2,
        ],
        compiler_params=pltpu.CompilerParams(use_tc_tiling_on_sc=False),
    )
    def rp(ep_h, out_h, buf_v, sems):
        wid = lax.axis_index("s") * NC + lax.axis_index("c")
        base0 = wid * EW

        def step(jo, carry):
            hs = []
            for k in range(2):
                b = base0 + (2 * jo + k) * RCH
                hs.append(pltpu.async_copy(
                    ep_h.at[pl.ds(b, RCH), pl.ds(0, EPS)], buf_v.at[k], sems[k]))
            ws = []
            for k in range(2):
                b = base0 + (2 * jo + k) * RCH
                hs[k].wait()
                ws.append(pltpu.async_copy(
                    buf_v.at[k], out_h.at[pl.ds(b, RCH)], sems[k]))
            for k in range(2):
                ws[k].wait()
            return carry

        lax.fori_loop(0, EW // (2 * RCH), step, 0)

    return rp(epack128)


def _sc_gather(t1, t2, dst, epack, atom_fea, slab):
    mesh = plsc.VectorSubcoreMesh(core_axis_name="c", subcore_axis_name="s")

    @functools.partial(
        pl.kernel,
        out_type=(
            jax.ShapeDtypeStruct((TSL, AF), jnp.float32),
            jax.ShapeDtypeStruct((TSL, AF), jnp.float32),
            jax.ShapeDtypeStruct((TSL,), jnp.int32),
        ),
        mesh=mesh,
        scratch_types=[
            pltpu.VMEM((UNR, CH), jnp.int32),
            pltpu.VMEM((UNR, CH), jnp.int32),
            pltpu.VMEM((UNR, CH), jnp.int32),
            pltpu.VMEM((UNR, CH, EPS), jnp.float32),
            pltpu.VMEM((UNR, CH, EPS), jnp.float32),
            pltpu.VMEM((UNR, CH, AF), jnp.float32),
            [pltpu.SemaphoreType.DMA] * UNR,
            [pltpu.SemaphoreType.DMA] * UNR,
        ],
        compiler_params=pltpu.CompilerParams(use_tc_tiling_on_sc=False),
    )
    def g(t1_h, t2_h, dst_h, ep_h, af_h, xa_o, epc_o, c_o,
          i1_v, i2_v, c_v, e1_v, e2_v, xa_v, sems_a, sems_b):
        wid = lax.axis_index("s") * NC + lax.axis_index("c")
        base0 = wid * BW
        rshift = slab * TSL

        def step(jo, carry):
            jbase = base0 + jo * (UNR * CH)
            h_idx = []
            for k in range(UNR):
                b = jbase + k * CH + rshift
                h1 = pltpu.async_copy(t1_h.at[pl.ds(b, CH)], i1_v.at[k], sems_a[k])
                h2 = pltpu.async_copy(t2_h.at[pl.ds(b, CH)], i2_v.at[k], sems_a[k])
                h_idx.append((h1, h2))
            h_c = []
            for k in range(UNR):
                h_idx[k][0].wait()
                h_idx[k][1].wait()
                h_c.append(pltpu.async_copy(dst_h.at[i1_v.at[k]], c_v.at[k], sems_a[k]))
            h_g = []
            for k in range(UNR):
                h_c[k].wait()
                ga = pltpu.async_copy(af_h.at[c_v.at[k]], xa_v.at[k], sems_a[k])
                g1 = pltpu.async_copy(ep_h.at[i1_v.at[k]], e1_v.at[k], sems_b[k])
                g2 = pltpu.async_copy(ep_h.at[i2_v.at[k]], e2_v.at[k], sems_b[k])
                h_g.append((ga, g1, g2))
            h_w = []
            for k in range(UNR):
                b = jbase + k * CH
                for h in h_g[k]:
                    h.wait()
                w0 = pltpu.async_copy(c_v.at[k], c_o.at[pl.ds(b, CH)], sems_a[k])
                w1 = pltpu.async_copy(xa_v.at[k], xa_o.at[pl.ds(b, CH)], sems_a[k])
                w2 = pltpu.async_copy(e1_v.at[k],
                                      epc_o.at[pl.ds(b, CH), pl.ds(0, EPS)],
                                      sems_b[k])
                w3 = pltpu.async_copy(e2_v.at[k],
                                      epc_o.at[pl.ds(b, CH), pl.ds(EPS, EPS)],
                                      sems_b[k])
                h_w.append((w0, w1, w2, w3))
            for k in range(UNR):
                for h in h_w[k]:
                    h.wait()
            return carry

        lax.fori_loop(0, ITERS // UNR, step, 0)

    return g(t1, t2, dst, epack, atom_fea)


def _stats_part_body(xa_r, epc_r, sperm_r, m1_r, m2_r,
                     maa_o, mar_o, mrr_o, sa_o, sr_o,
                     maa, mar, mrr, sa, sr):
    i = pl.program_id(0)

    @pl.when(i == 0)
    def _():
        maa[...] = jnp.zeros_like(maa)
        mar[...] = jnp.zeros_like(mar)
        mrr[...] = jnp.zeros_like(mrr)
        sa[...] = jnp.zeros_like(sa)
        sr[...] = jnp.zeros_like(sr)

    xa_b = xa_r[...]
    bz, ang = _bz_ang(epc_r[...], sperm_r[...], m1_r[...], m2_r[...])
    xr_b = jnp.concatenate([bz[:, 0:16], bz[:, 24:40], ang], axis=1)
    dn = (((0,), (0,)), ((), ()))
    maa[...] += lax.dot_general(xa_b, xa_b, dn, preferred_element_type=jnp.float32)
    mar[...] += lax.dot_general(xa_b, xr_b, dn, preferred_element_type=jnp.float32)
    mrr[...] += lax.dot_general(xr_b, xr_b, dn, preferred_element_type=jnp.float32)
    sa[...] += jnp.sum(xa_b, axis=0, keepdims=True)
    sr[...] += jnp.sum(xr_b, axis=0, keepdims=True)

    @pl.when(i == GT - 1)
    def _():
        maa_o[...] = maa[...]
        mar_o[...] = mar[...]
        mrr_o[...] = mrr[...]
        sa_o[...] = sa[...]
        sr_o[...] = sr[...]


_M_SHAPES = [(AF, AF), (AF, 48), (48, 48), (1, AF), (1, 48)]


def _stats_part(xa, epc, sperm, m1, m2):
    return pl.pallas_call(
        _stats_part_body,
        grid=(GT,),
        in_specs=[
            pl.BlockSpec((BT, AF), lambda i: (i, 0)),
            pl.BlockSpec((BT, AF), lambda i: (i, 0)),
            pl.BlockSpec((AF, AF), lambda i: (0, 0)),
            pl.BlockSpec((AF, 16), lambda i: (0, 0)),
            pl.BlockSpec((AF, 16), lambda i: (0, 0)),
        ],
        out_specs=[pl.BlockSpec(s, lambda i: (0, 0)) for s in _M_SHAPES],
        out_shape=[jax.ShapeDtypeStruct(s, jnp.float32) for s in _M_SHAPES],
        scratch_shapes=[pltpu.VMEM(s, jnp.float32) for s in _M_SHAPES],
    )(xa, epc, sperm, m1, m2)


def _stats_fin_body(maa0, mar0, mrr0, sa0, sr0, maa1, mar1, mrr1, sa1, sr1,
                    wt_r, wa_r, we_r, wg_r, b_r, g1_r, be1_r,
                    wsa_o, wse_o, wsg_o, bs_o):
    maa = maa0[...] + maa1[...]
    mar = mar0[...] + mar1[...]
    mrr = mrr0[...] + mrr1[...]
    sa = sa0[...] + sa1[...]
    sr = sr0[...] + sr1[...]
    wt = wt_r[...]
    wa = wt[:AF, :]
    wr = wt[AF:, :]
    dn0 = (((0,), (0,)), ((), ()))
    z_top = (jnp.dot(maa, wa, preferred_element_type=jnp.float32)
             + jnp.dot(mar, wr, preferred_element_type=jnp.float32))
    z_bot = (lax.dot_general(mar, wa, dn0, preferred_element_type=jnp.float32)
             + jnp.dot(mrr, wr, preferred_element_type=jnp.float32))
    sw = (jnp.dot(sa, wa, preferred_element_type=jnp.float32)
          + jnp.dot(sr, wr, preferred_element_type=jnp.float32))
    bvec = b_r[...]
    tf = jnp.float32(T)
    e2 = (jnp.sum(wa * z_top, axis=0, keepdims=True)
          + jnp.sum(wr * z_bot, axis=0, keepdims=True)
          + 2.0 * bvec * sw + tf * bvec * bvec)
    mean = sw / tf + bvec
    var = e2 / tf - mean * mean
    s1 = g1_r[...] / jnp.sqrt(var + 1e-5)
    t1 = be1_r[...] - mean * s1
    wsa_o[...] = wa_r[...] * s1
    wse_o[...] = we_r[...] * s1
    wsg_o[...] = wg_r[...] * s1
    bs_o[...] = bvec * s1 + t1


def _stats_fin(m0, m1s, wt, wa, we, wg, b2d, g12d, be12d):
    full = lambda s: pl.BlockSpec(s, lambda: (0, 0))
    return pl.pallas_call(
        _stats_fin_body,
        in_specs=([full(s) for s in _M_SHAPES] + [full(s) for s in _M_SHAPES]
                  + [full((IN_DIM, OUT1)), full((AF, OUT1)), full((AF, OUT1)),
                     full((16, OUT1)), full((1, OUT1)), full((1, OUT1)),
                     full((1, OUT1))]),
        out_specs=[full((AF, OUT1)), full((AF, OUT1)), full((16, OUT1)),
                   full((1, OUT1))],
        out_shape=[
            jax.ShapeDtypeStruct((AF, OUT1), jnp.float32),
            jax.ShapeDtypeStruct((AF, OUT1), jnp.float32),
            jax.ShapeDtypeStruct((16, OUT1), jnp.float32),
            jax.ShapeDtypeStruct((1, OUT1), jnp.float32),
        ],
    )(*m0, *m1s, wt, wa, we, wg, b2d, g12d, be12d)


def _fc_body(xa_r, epc_r, sperm_r, m1_r, m2_r, wsa_r, wse_r, wsg_r, bs_r, msg_o):
    xa_b = xa_r[...]
    bz, ang = _bz_ang(epc_r[...], sperm_r[...], m1_r[...], m2_r[...])
    y = (jnp.dot(xa_b, wsa_r[...], preferred_element_type=jnp.float32)
         + jnp.dot(bz, wse_r[...], preferred_element_type=jnp.float32)
         + jnp.dot(ang, wsg_r[...], preferred_element_type=jnp.float32)
         + bs_r[...])
    gate = y[:, :AF]
    core = y[:, AF:]
    sig = 1.0 / (1.0 + jnp.exp(-gate))
    sp = jnp.maximum(core, 0.0) + jnp.log1p(jnp.exp(-jnp.abs(core)))
    msg_o[...] = sig * sp


def _fc(xa, epc, sperm, m1, m2, wsa, wse, wsg, bs):
    return pl.pallas_call(
        _fc_body,
        grid=(GT,),
        in_specs=[
            pl.BlockSpec((BT, AF), lambda i: (i, 0)),
            pl.BlockSpec((BT, AF), lambda i: (i, 0)),
            pl.BlockSpec((AF, AF), lambda i: (0, 0)),
            pl.BlockSpec((AF, 16), lambda i: (0, 0)),
            pl.BlockSpec((AF, 16), lambda i: (0, 0)),
            pl.BlockSpec((AF, OUT1), lambda i: (0, 0)),
            pl.BlockSpec((AF, OUT1), lambda i: (0, 0)),
            pl.BlockSpec((16, OUT1), lambda i: (0, 0)),
            pl.BlockSpec((1, OUT1), lambda i: (0, 0)),
        ],
        out_specs=pl.BlockSpec((BT, AF), lambda i: (i, 0)),
        out_shape=jax.ShapeDtypeStruct((TSL, AF), jnp.float32),
    )(xa, epc, sperm, m1, m2, wsa, wse, wsg, bs)


def _sc_scatter(msg, cidx, zeros):
    mesh = plsc.VectorSubcoreMesh(core_axis_name="c", subcore_axis_name="s")

    HAF = AF // 2
    BWS = TSL // NS
    ITERS_S = BWS // CH

    @functools.partial(
        pl.kernel,
        out_type=jax.ShapeDtypeStruct((N, AF), jnp.float32),
        mesh=mesh,
        scratch_types=[
            pltpu.VMEM((UNR, CH), jnp.int32),
            pltpu.VMEM((UNR, CH, HAF), jnp.float32),
            pltpu.VMEM_SHARED((N, HAF), jnp.float32),
            [pltpu.SemaphoreType.DMA] * UNR,
            [pltpu.SemaphoreType.DMA] * UNR,
        ],
        compiler_params=pltpu.CompilerParams(use_tc_tiling_on_sc=False),
    )
    def r(msg_h, c_h, z_h, out_h, idx_v, msg_v, acc_sh, sems_l, sems_s):
        cid = lax.axis_index("c")
        sid = lax.axis_index("s")
        col0 = cid * HAF
        pltpu.sync_copy(z_h.at[pl.ds(sid * ROWS_PER_TILE, ROWS_PER_TILE)],
                        acc_sh.at[pl.ds(sid * ROWS_PER_TILE, ROWS_PER_TILE)])
        plsc.subcore_barrier()
        base0 = sid * BWS

        def step(jo, carry):
            jbase = base0 + jo * (UNR * CH)
            h_l = []
            for k in range(UNR):
                b = jbase + k * CH
                l0 = pltpu.async_copy(c_h.at[pl.ds(b, CH)], idx_v.at[k], sems_l[k])
                l1 = pltpu.async_copy(msg_h.at[pl.ds(b, CH), pl.ds(col0, HAF)],
                                      msg_v.at[k], sems_l[k])
                h_l.append((l0, l1))
            h_s = []
            for k in range(UNR):
                h_l[k][0].wait()
                h_l[k][1].wait()
                h_s.append(pltpu.async_copy(
                    msg_v.at[k], acc_sh.at[idx_v.at[k]], sems_s[k], add=True))
            for k in range(UNR):
                h_s[k].wait()
            return carry

        lax.fori_loop(0, ITERS_S // UNR, step, 0)
        plsc.subcore_barrier()
        pltpu.sync_copy(
            acc_sh.at[pl.ds(sid * ROWS_PER_TILE, ROWS_PER_TILE)],
            out_h.at[pl.ds(sid * ROWS_PER_TILE, ROWS_PER_TILE), pl.ds(col0, HAF)])

    return r(msg, cidx, zeros)


def _final_body(a0_r, a1_r, af_r, w2_r, b2_r, out_r):
    agg = a0_r[...] + a1_r[...]
    m = jnp.mean(agg, axis=0, keepdims=True)
    d = agg - m
    v = jnp.mean(d * d, axis=0, keepdims=True)
    nrm = d / jnp.sqrt(v + 1e-5) * w2_r[...] + b2_r[...]
    x = af_r[...] + nrm
    out_r[...] = jnp.maximum(x, 0.0) + jnp.log1p(jnp.exp(-jnp.abs(x)))


def _final(a0, a1, atom_fea, w22d, b22d):
    return pl.pallas_call(
        _final_body,
        grid=(1,),
        in_specs=[
            pl.BlockSpec((N, AF), lambda i: (0, 0)),
            pl.BlockSpec((N, AF), lambda i: (0, 0)),
            pl.BlockSpec((N, AF), lambda i: (0, 0)),
            pl.BlockSpec((1, AF), lambda i: (0, 0)),
            pl.BlockSpec((1, AF), lambda i: (0, 0)),
        ],
        out_specs=pl.BlockSpec((N, AF), lambda i: (0, 0)),
        out_shape=jax.ShapeDtypeStruct((N, AF), jnp.float32),
    )(a0, a1, atom_fea, w22d, b22d)


def kernel(atom_fea, edge_fea, r_ij, dist, edge_index, triplet_idx,
           W_fc, b_fc, bn1_w, bn1_b, bn2_w, bn2_b):
    t1 = triplet_idx[0].astype(jnp.int32)
    t2 = triplet_idx[1].astype(jnp.int32)
    dst = edge_index[1].astype(jnp.int32)
    wt = jnp.transpose(W_fc)
    b2d = b_fc.reshape(1, OUT1)
    g12d = bn1_w.reshape(1, OUT1)
    be12d = bn1_b.reshape(1, OUT1)
    w22d = bn2_w.reshape(1, AF)
    b22d = bn2_b.reshape(1, AF)
    zeros = jnp.zeros((N, AF // 2), jnp.float32)

    epack128 = _prep(jnp.transpose(edge_fea), jnp.transpose(r_ij),
                     dist.reshape(1, E))
    sperm_np = np.zeros((AF, AF), np.float32)
    for i in range(4):
        sperm_np[40 + i, 16 + i] = 1.0
    m1_np = np.zeros((AF, 16), np.float32)
    m1_np[16:19, :] = 1.0
    m2_np = np.zeros((AF, 16), np.float32)
    m2_np[19, :] = 1.0
    sperm = jnp.asarray(sperm_np)
    m1 = jnp.asarray(m1_np)
    m2 = jnp.asarray(m2_np)
    wa = wt[:AF, :]
    we = (jnp.zeros((AF, OUT1), jnp.float32)
          .at[0:16].set(wt[AF:AF + 16, :])
          .at[24:40].set(wt[AF + 16:AF + 32, :]))
    wg = wt[AF + 32:, :]

    ep24 = _sc_repack(epack128)
    slabs = [_sc_gather(t1, t2, dst, ep24, atom_fea, s) for s in range(SLABS)]
    moments = [_stats_part(xa, epc, sperm, m1, m2) for xa, epc, _ in slabs]
    wsa, wse, wsg, bs = _stats_fin(moments[0], moments[1], wt, wa, we, wg,
                                   b2d, g12d, be12d)
    aggs = []
    for xa, epc, cidx in slabs:
        msg = _fc(xa, epc, sperm, m1, m2, wsa, wse, wsg, bs)
        aggs.append(_sc_scatter(msg, cidx, zeros))
    return _final(aggs[0], aggs[1], atom_fea, w22d, b22d)


# speedup vs baseline: 47.1190x; 1.0490x over previous
import functools

import jax
import jax.numpy as jnp
import numpy as np
from jax import lax
from jax.experimental import pallas as pl
from jax.experimental.pallas import tpu as pltpu
from jax.experimental.pallas import tpu_sc as plsc

N = 10000
E = 320000
T = 640000
AF = 128
EPW = 20
EPS = 24
OUT1 = 256
IN_DIM = 176

NC = 2
NS = 16
NW = NC * NS
SLABS = 4
TSL = T // SLABS
BW = TSL // NW
CH = 40
ITERS = BW // CH
UNR = 5
CHS = 80

BT = 3200
GT = TSL // BT
BE = 3200
ROWS_PER_TILE = N // NS


def _bz_ang(b, sperm, m1, m2):
    lanes = lax.broadcasted_iota(jnp.int32, (1, 128), 1)
    bz = jnp.where(lanes < 48, b, 0.0)
    bshift = jnp.dot(bz, sperm, preferred_element_type=jnp.float32)
    p = bz * bshift
    num = jnp.dot(p, m1, preferred_element_type=jnp.float32)
    den = jnp.dot(p, m2, preferred_element_type=jnp.float32)
    cos = jnp.clip(num / jnp.maximum(den, 1e-16), -1.0, 1.0)
    centers = (lax.broadcasted_iota(jnp.int32, (1, 16), 1).astype(jnp.float32)
               * (2.0 / 15.0) - 1.0)
    ang = jnp.exp(-((cos - centers) ** 2) / (0.15 ** 2))
    return bz, ang


def _prep_body(eft_r, rt_r, dt_r, out_r):
    stack = jnp.concatenate(
        [eft_r[...], rt_r[...], dt_r[...], jnp.zeros((4, BE), jnp.float32)],
        axis=0)
    t = jnp.transpose(stack)
    out_r[...] = jnp.concatenate([t, jnp.zeros((BE, AF - EPS), jnp.float32)],
                                 axis=1)


def _prep(eft, rt, dt):
    return pl.pallas_call(
        _prep_body,
        grid=(E // BE,),
        in_specs=[
            pl.BlockSpec((16, BE), lambda i: (0, i)),
            pl.BlockSpec((3, BE), lambda i: (0, i)),
            pl.BlockSpec((1, BE), lambda i: (0, i)),
        ],
        out_specs=pl.BlockSpec((BE, AF), lambda i: (i, 0)),
        out_shape=jax.ShapeDtypeStruct((E, AF), jnp.float32),
    )(eft, rt, dt)


def _sc_repack(epack128):
    mesh = plsc.VectorSubcoreMesh(core_axis_name="c", subcore_axis_name="s")
    EW = E // NW
    RCH = 1000

    @functools.partial(
        pl.kernel,
        out_type=jax.ShapeDtypeStruct((E, EPS), jnp.float32),
        mesh=mesh,
        scratch_types=[
            pltpu.VMEM((2, RCH, EPS), jnp.float32),
            [pltpu.SemaphoreType.DMA] * 2,
        ],
        compiler_params=pltpu.CompilerParams(use_tc_tiling_on_sc=False),
    )
    def rp(ep_h, out_h, buf_v, sems):
        wid = lax.axis_index("s") * NC + lax.axis_index("c")
        base0 = wid * EW

        def step(jo, carry):
            hs = []
            for k in range(2):
                b = base0 + (2 * jo + k) * RCH
                hs.append(pltpu.async_copy(
                    ep_h.at[pl.ds(b, RCH), pl.ds(0, EPS)], buf_v.at[k], sems[k]))
            ws = []
            for k in range(2):
                b = base0 + (2 * jo + k) * RCH
                hs[k].wait()
                ws.append(pltpu.async_copy(
                    buf_v.at[k], out_h.at[pl.ds(b, RCH)], sems[k]))
            for k in range(2):
                ws[k].wait()
            return carry

        lax.fori_loop(0, EW // (2 * RCH), step, 0)

    return rp(epack128)


def _sc_gather(t1, t2, dst, epack, atom_fea, slab):
    mesh = plsc.VectorSubcoreMesh(core_axis_name="c", subcore_axis_name="s")

    @functools.partial(
        pl.kernel,
        out_type=(
            jax.ShapeDtypeStruct((TSL, AF), jnp.float32),
            jax.ShapeDtypeStruct((TSL, AF), jnp.float32),
            jax.ShapeDtypeStruct((TSL,), jnp.int32),
        ),
        mesh=mesh,
        scratch_types=[
            pltpu.VMEM((UNR, CH), jnp.int32),
            pltpu.VMEM((UNR, CH), jnp.int32),
            pltpu.VMEM((UNR, CH), jnp.int32),
            pltpu.VMEM((UNR, CH, EPS), jnp.float32),
            pltpu.VMEM((UNR, CH, EPS), jnp.float32),
            pltpu.VMEM((UNR, CH, AF), jnp.float32),
            [pltpu.SemaphoreType.DMA] * UNR,
            [pltpu.SemaphoreType.DMA] * UNR,
        ],
        compiler_params=pltpu.CompilerParams(use_tc_tiling_on_sc=False),
    )
    def g(t1_h, t2_h, dst_h, ep_h, af_h, xa_o, epc_o, c_o,
          i1_v, i2_v, c_v, e1_v, e2_v, xa_v, sems_a, sems_b):
        wid = lax.axis_index("s") * NC + lax.axis_index("c")
        base0 = wid * BW
        rshift = slab * TSL

        def step(jo, carry):
            jbase = base0 + jo * (UNR * CH)
            h_idx = []
            for k in range(UNR):
                b = jbase + k * CH + rshift
                h1 = pltpu.async_copy(t1_h.at[pl.ds(b, CH)], i1_v.at[k], sems_a[k])
                h2 = pltpu.async_copy(t2_h.at[pl.ds(b, CH)], i2_v.at[k], sems_a[k])
                h_idx.append((h1, h2))
            h_c = []
            for k in range(UNR):
                h_idx[k][0].wait()
                h_idx[k][1].wait()
                h_c.append(pltpu.async_copy(dst_h.at[i1_v.at[k]], c_v.at[k], sems_a[k]))
            h_g = []
            for k in range(UNR):
                h_c[k].wait()
                ga = pltpu.async_copy(af_h.at[c_v.at[k]], xa_v.at[k], sems_a[k])
                g1 = pltpu.async_copy(ep_h.at[i1_v.at[k]], e1_v.at[k], sems_b[k])
                g2 = pltpu.async_copy(ep_h.at[i2_v.at[k]], e2_v.at[k], sems_b[k])
                h_g.append((ga, g1, g2))
            h_w = []
            for k in range(UNR):
                b = jbase + k * CH
                for h in h_g[k]:
                    h.wait()
                w0 = pltpu.async_copy(c_v.at[k], c_o.at[pl.ds(b, CH)], sems_a[k])
                w1 = pltpu.async_copy(xa_v.at[k], xa_o.at[pl.ds(b, CH)], sems_a[k])
                w2 = pltpu.async_copy(e1_v.at[k],
                                      epc_o.at[pl.ds(b, CH), pl.ds(0, EPS)],
                                      sems_b[k])
                w3 = pltpu.async_copy(e2_v.at[k],
                                      epc_o.at[pl.ds(b, CH), pl.ds(EPS, EPS)],
                                      sems_b[k])
                h_w.append((w0, w1, w2, w3))
            for k in range(UNR):
                for h in h_w[k]:
                    h.wait()
            return carry

        lax.fori_loop(0, ITERS // UNR, step, 0)

    return g(t1, t2, dst, epack, atom_fea)


def _stats_part_body(xa_r, epc_r, sperm_r, m1_r, m2_r,
                     maa_o, mar_o, mrr_o, sa_o, sr_o,
                     maa, mar, mrr, sa, sr):
    i = pl.program_id(0)

    @pl.when(i == 0)
    def _():
        maa[...] = jnp.zeros_like(maa)
        mar[...] = jnp.zeros_like(mar)
        mrr[...] = jnp.zeros_like(mrr)
        sa[...] = jnp.zeros_like(sa)
        sr[...] = jnp.zeros_like(sr)

    xa_b = xa_r[...]
    bz, ang = _bz_ang(epc_r[...], sperm_r[...], m1_r[...], m2_r[...])
    xr_b = jnp.concatenate([bz[:, 0:16], bz[:, 24:40], ang], axis=1)
    dn = (((0,), (0,)), ((), ()))
    maa[...] += lax.dot_general(xa_b, xa_b, dn, preferred_element_type=jnp.float32)
    mar[...] += lax.dot_general(xa_b, xr_b, dn, preferred_element_type=jnp.float32)
    mrr[...] += lax.dot_general(xr_b, xr_b, dn, preferred_element_type=jnp.float32)
    sa[...] += jnp.sum(xa_b, axis=0, keepdims=True)
    sr[...] += jnp.sum(xr_b, axis=0, keepdims=True)

    @pl.when(i == GT - 1)
    def _():
        maa_o[...] = maa[...]
        mar_o[...] = mar[...]
        mrr_o[...] = mrr[...]
        sa_o[...] = sa[...]
        sr_o[...] = sr[...]


_M_SHAPES = [(AF, AF), (AF, 48), (48, 48), (1, AF), (1, 48)]


def _stats_part(xa, epc, sperm, m1, m2):
    return pl.pallas_call(
        _stats_part_body,
        grid=(GT,),
        in_specs=[
            pl.BlockSpec((BT, AF), lambda i: (i, 0)),
            pl.BlockSpec((BT, AF), lambda i: (i, 0)),
            pl.BlockSpec((AF, AF), lambda i: (0, 0)),
            pl.BlockSpec((AF, 16), lambda i: (0, 0)),
            pl.BlockSpec((AF, 16), lambda i: (0, 0)),
        ],
        out_specs=[pl.BlockSpec(s, lambda i: (0, 0)) for s in _M_SHAPES],
        out_shape=[jax.ShapeDtypeStruct(s, jnp.float32) for s in _M_SHAPES],
        scratch_shapes=[pltpu.VMEM(s, jnp.float32) for s in _M_SHAPES],
    )(xa, epc, sperm, m1, m2)


def _stats_fin_body(*refs):
    nm = 5 * SLABS
    mrefs = refs[:nm]
    (wt_r, wa_r, we_r, wg_r, b_r, g1_r, be1_r,
     wsa_o, wse_o, wsg_o, bs_o) = refs[nm:]
    maa, mar, mrr, sa, sr = (
        sum(mrefs[s * 5 + i][...] for s in range(SLABS)) for i in range(5))
    wt = wt_r[...]
    wa = wt[:AF, :]
    wr = wt[AF:, :]
    dn0 = (((0,), (0,)), ((), ()))
    z_top = (jnp.dot(maa, wa, preferred_element_type=jnp.float32)
             + jnp.dot(mar, wr, preferred_element_type=jnp.float32))
    z_bot = (lax.dot_general(mar, wa, dn0, preferred_element_type=jnp.float32)
             + jnp.dot(mrr, wr, preferred_element_type=jnp.float32))
    sw = (jnp.dot(sa, wa, preferred_element_type=jnp.float32)
          + jnp.dot(sr, wr, preferred_element_type=jnp.float32))
    bvec = b_r[...]
    tf = jnp.float32(T)
    e2 = (jnp.sum(wa * z_top, axis=0, keepdims=True)
          + jnp.sum(wr * z_bot, axis=0, keepdims=True)
          + 2.0 * bvec * sw + tf * bvec * bvec)
    mean = sw / tf + bvec
    var = e2 / tf - mean * mean
    s1 = g1_r[...] / jnp.sqrt(var + 1e-5)
    t1 = be1_r[...] - mean * s1
    wsa_o[...] = wa_r[...] * s1
    wse_o[...] = we_r[...] * s1
    wsg_o[...] = wg_r[...] * s1
    bs_o[...] = bvec * s1 + t1


def _stats_fin(moments, wt, wa, we, wg, b2d, g12d, be12d):
    full = lambda s: pl.BlockSpec(s, lambda: (0, 0))
    return pl.pallas_call(
        _stats_fin_body,
        in_specs=([full(s) for s in _M_SHAPES] * SLABS
                  + [full((IN_DIM, OUT1)), full((AF, OUT1)), full((AF, OUT1)),
                     full((16, OUT1)), full((1, OUT1)), full((1, OUT1)),
                     full((1, OUT1))]),
        out_specs=[full((AF, OUT1)), full((AF, OUT1)), full((16, OUT1)),
                   full((1, OUT1))],
        out_shape=[
            jax.ShapeDtypeStruct((AF, OUT1), jnp.float32),
            jax.ShapeDtypeStruct((AF, OUT1), jnp.float32),
            jax.ShapeDtypeStruct((16, OUT1), jnp.float32),
            jax.ShapeDtypeStruct((1, OUT1), jnp.float32),
        ],
    )(*[m for mom in moments for m in mom], wt, wa, we, wg, b2d, g12d, be12d)


def _fc_body(xa_r, epc_r, sperm_r, m1_r, m2_r, wsa_r, wse_r, wsg_r, bs_r, msg_o):
    xa_b = xa_r[...]
    bz, ang = _bz_ang(epc_r[...], sperm_r[...], m1_r[...], m2_r[...])
    y = (jnp.dot(xa_b, wsa_r[...], preferred_element_type=jnp.float32)
         + jnp.dot(bz, wse_r[...], preferred_element_type=jnp.float32)
         + jnp.dot(ang, wsg_r[...], preferred_element_type=jnp.float32)
         + bs_r[...])
    gate = y[:, :AF]
    core = y[:, AF:]
    sig = 1.0 / (1.0 + jnp.exp(-gate))
    sp = jnp.maximum(core, 0.0) + jnp.log1p(jnp.exp(-jnp.abs(core)))
    msg_o[...] = sig * sp


def _fc(xa, epc, sperm, m1, m2, wsa, wse, wsg, bs):
    return pl.pallas_call(
        _fc_body,
        grid=(GT,),
        in_specs=[
            pl.BlockSpec((BT, AF), lambda i: (i, 0)),
            pl.BlockSpec((BT, AF), lambda i: (i, 0)),
            pl.BlockSpec((AF, AF), lambda i: (0, 0)),
            pl.BlockSpec((AF, 16), lambda i: (0, 0)),
            pl.BlockSpec((AF, 16), lambda i: (0, 0)),
            pl.BlockSpec((AF, OUT1), lambda i: (0, 0)),
            pl.BlockSpec((AF, OUT1), lambda i: (0, 0)),
            pl.BlockSpec((16, OUT1), lambda i: (0, 0)),
            pl.BlockSpec((1, OUT1), lambda i: (0, 0)),
        ],
        out_specs=pl.BlockSpec((BT, AF), lambda i: (i, 0)),
        out_shape=jax.ShapeDtypeStruct((TSL, AF), jnp.float32),
    )(xa, epc, sperm, m1, m2, wsa, wse, wsg, bs)


def _sc_scatter(msg, cidx, zeros):
    mesh = plsc.VectorSubcoreMesh(core_axis_name="c", subcore_axis_name="s")

    HAF = AF // 2
    BWS = TSL // NS
    ITERS_S = BWS // CHS

    @functools.partial(
        pl.kernel,
        out_type=jax.ShapeDtypeStruct((N, AF), jnp.float32),
        mesh=mesh,
        scratch_types=[
            pltpu.VMEM((UNR, CHS), jnp.int32),
            pltpu.VMEM((UNR, CHS, HAF), jnp.float32),
            pltpu.VMEM_SHARED((N, HAF), jnp.float32),
            [pltpu.SemaphoreType.DMA] * UNR,
            [pltpu.SemaphoreType.DMA] * UNR,
        ],
        compiler_params=pltpu.CompilerParams(use_tc_tiling_on_sc=False),
    )
    def r(msg_h, c_h, z_h, out_h, idx_v, msg_v, acc_sh, sems_l, sems_s):
        cid = lax.axis_index("c")
        sid = lax.axis_index("s")
        col0 = cid * HAF
        pltpu.sync_copy(z_h.at[pl.ds(sid * ROWS_PER_TILE, ROWS_PER_TILE)],
                        acc_sh.at[pl.ds(sid * ROWS_PER_TILE, ROWS_PER_TILE)])
        plsc.subcore_barrier()
        base0 = sid * BWS

        def step(jo, carry):
            jbase = base0 + jo * (UNR * CHS)
            h_l = []
            for k in range(UNR):
                b = jbase + k * CHS
                l0 = pltpu.async_copy(c_h.at[pl.ds(b, CHS)], idx_v.at[k], sems_l[k])
                l1 = pltpu.async_copy(msg_h.at[pl.ds(b, CHS), pl.ds(col0, HAF)],
                                      msg_v.at[k], sems_l[k])
                h_l.append((l0, l1))
            h_s = []
            for k in range(UNR):
                h_l[k][0].wait()
                h_l[k][1].wait()
                h_s.append(pltpu.async_copy(
                    msg_v.at[k], acc_sh.at[idx_v.at[k]], sems_s[k], add=True))
            for k in range(UNR):
                h_s[k].wait()
            return carry

        lax.fori_loop(0, ITERS_S // UNR, step, 0)
        plsc.subcore_barrier()
        pltpu.sync_copy(
            acc_sh.at[pl.ds(sid * ROWS_PER_TILE, ROWS_PER_TILE)],
            out_h.at[pl.ds(sid * ROWS_PER_TILE, ROWS_PER_TILE), pl.ds(col0, HAF)])

    return r(msg, cidx, zeros)


def _final_body(*refs):
    aggs = refs[:SLABS]
    af_r, w2_r, b2_r, out_r = refs[SLABS:]
    agg = aggs[0][...]
    for a in aggs[1:]:
        agg = agg + a[...]
    m = jnp.mean(agg, axis=0, keepdims=True)
    d = agg - m
    v = jnp.mean(d * d, axis=0, keepdims=True)
    nrm = d / jnp.sqrt(v + 1e-5) * w2_r[...] + b2_r[...]
    x = af_r[...] + nrm
    out_r[...] = jnp.maximum(x, 0.0) + jnp.log1p(jnp.exp(-jnp.abs(x)))


def _final(aggs, atom_fea, w22d, b22d):
    return pl.pallas_call(
        _final_body,
        grid=(1,),
        in_specs=([pl.BlockSpec((N, AF), lambda i: (0, 0))] * (SLABS + 1)
                  + [pl.BlockSpec((1, AF), lambda i: (0, 0)),
                     pl.BlockSpec((1, AF), lambda i: (0, 0))]),
        out_specs=pl.BlockSpec((N, AF), lambda i: (0, 0)),
        out_shape=jax.ShapeDtypeStruct((N, AF), jnp.float32),
    )(*aggs, atom_fea, w22d, b22d)


def kernel(atom_fea, edge_fea, r_ij, dist, edge_index, triplet_idx,
           W_fc, b_fc, bn1_w, bn1_b, bn2_w, bn2_b):
    t1 = triplet_idx[0].astype(jnp.int32)
    t2 = triplet_idx[1].astype(jnp.int32)
    dst = edge_index[1].astype(jnp.int32)
    wt = jnp.transpose(W_fc)
    b2d = b_fc.reshape(1, OUT1)
    g12d = bn1_w.reshape(1, OUT1)
    be12d = bn1_b.reshape(1, OUT1)
    w22d = bn2_w.reshape(1, AF)
    b22d = bn2_b.reshape(1, AF)
    zeros = jnp.zeros((N, AF // 2), jnp.float32)

    epack128 = _prep(jnp.transpose(edge_fea), jnp.transpose(r_ij),
                     dist.reshape(1, E))
    sperm_np = np.zeros((AF, AF), np.float32)
    for i in range(4):
        sperm_np[40 + i, 16 + i] = 1.0
    m1_np = np.zeros((AF, 16), np.float32)
    m1_np[16:19, :] = 1.0
    m2_np = np.zeros((AF, 16), np.float32)
    m2_np[19, :] = 1.0
    sperm = jnp.asarray(sperm_np)
    m1 = jnp.asarray(m1_np)
    m2 = jnp.asarray(m2_np)
    wa = wt[:AF, :]
    we = (jnp.zeros((AF, OUT1), jnp.float32)
          .at[0:16].set(wt[AF:AF + 16, :])
          .at[24:40].set(wt[AF + 16:AF + 32, :]))
    wg = wt[AF + 32:, :]

    ep24 = _sc_repack(epack128)
    slabs = [_sc_gather(t1, t2, dst, ep24, atom_fea, s) for s in range(SLABS)]
    moments = [_stats_part(xa, epc, sperm, m1, m2) for xa, epc, _ in slabs]
    wsa, wse, wsg, bs = _stats_fin(moments, wt, wa, we, wg,
                                   b2d, g12d, be12d)
    aggs = []
    for xa, epc, cidx in slabs:
        msg = _fc(xa, epc, sperm, m1, m2, wsa, wse, wsg, bs)
        aggs.append(_sc_scatter(msg, cidx, zeros))
    return _final(aggs, atom_fea, w22d, b22d)
